# Initial kernel scaffold; baseline (speedup 1.0000x reference)
import jax
import jax.numpy as jnp
from jax.experimental import pallas as pl


def kernel(x, edge_index, edge_attr, same_frame_edge_index, same_frame_edge_attr, Wff1, bff1, Wff2, bff2, Wfb1, bfb1, Wfb2, bfb2, Wfr1, bfr1, Wfr2, bfr2, Wt1, bt1):
    raise NotImplementedError("write your pallas kernel here")


# speedup vs baseline: 2.2771x; 2.2771x over previous
import functools

import jax
import jax.numpy as jnp
from jax import lax
from jax.experimental import pallas as pl
from jax.experimental.pallas import tpu as pltpu
from jax.experimental.pallas import tpu_sc as plsc

N = 10000
E = 320000
ESF = 160000
D = 128
H1 = 32
H2 = 16

NC = 2
NS = 16
NW = NC * NS

EPW = E // NW
FPW = ESF // NW
CH_T = 80
CH_F = 40
NT = EPW // CH_T
NF = FPW // CH_F

ROWS_PER_SUB = N // NS


def _proj_body(x_ref, w_ref, tf_ref, tp_ref, te_ref, tl_ref):
    p = jnp.dot(x_ref[...], w_ref[...], preferred_element_type=jnp.float32)
    tf_ref[...] = p[:, 0:64]
    tp_ref[...] = p[:, 64:128]
    te_ref[...] = p[:, 128:160]
    tl_ref[...] = p[:, 160:192]


def _proj(x, wcat):
    bn = 2000
    return pl.pallas_call(
        _proj_body,
        grid=(N // bn,),
        in_specs=[
            pl.BlockSpec((bn, D), lambda i: (i, 0)),
            pl.BlockSpec((D, 192), lambda i: (0, 0)),
        ],
        out_specs=[
            pl.BlockSpec((bn, 64), lambda i: (i, 0)),
            pl.BlockSpec((bn, 64), lambda i: (i, 0)),
            pl.BlockSpec((bn, 32), lambda i: (i, 0)),
            pl.BlockSpec((bn, 32), lambda i: (i, 0)),
        ],
        out_shape=[
            jax.ShapeDtypeStruct((N, 64), jnp.float32),
            jax.ShapeDtypeStruct((N, 64), jnp.float32),
            jax.ShapeDtypeStruct((N, 32), jnp.float32),
            jax.ShapeDtypeStruct((N, 32), jnp.float32),
        ],
    )(x, wcat)


def _sc_gather_body(tf, tp, te, tl, fut, past, early, later,
                    g_fut, g_past, g_early, g_later,
                    idx_t, rows_t, idx_f, rows_f, sem):
    wid = lax.axis_index("s") * NC + lax.axis_index("c")

    bt = wid * EPW

    def t_step(i, _):
        b = bt + i * CH_T
        pltpu.sync_copy(fut.at[pl.ds(b, CH_T)], idx_t)
        pltpu.async_copy(tf.at[idx_t], rows_t, sem).wait()
        pltpu.sync_copy(rows_t, g_fut.at[pl.ds(b, CH_T)])
        pltpu.sync_copy(past.at[pl.ds(b, CH_T)], idx_t)
        pltpu.async_copy(tp.at[idx_t], rows_t, sem).wait()
        pltpu.sync_copy(rows_t, g_past.at[pl.ds(b, CH_T)])
        return 0

    lax.fori_loop(0, NT, t_step, 0)

    bf = wid * FPW

    def f_step(i, _):
        b = bf + i * CH_F
        pltpu.sync_copy(early.at[pl.ds(b, CH_F)], idx_f)
        pltpu.async_copy(te.at[idx_f], rows_f, sem).wait()
        pltpu.sync_copy(rows_f, g_early.at[pl.ds(b, CH_F)])
        pltpu.sync_copy(later.at[pl.ds(b, CH_F)], idx_f)
        pltpu.async_copy(tl.at[idx_f], rows_f, sem).wait()
        pltpu.sync_copy(rows_f, g_later.at[pl.ds(b, CH_F)])
        return 0

    lax.fori_loop(0, NF, f_step, 0)


@functools.cache
def _sc_gather():
    return pl.kernel(
        _sc_gather_body,
        out_type=(
            jax.ShapeDtypeStruct((E, 64), jnp.float32),
            jax.ShapeDtypeStruct((E, 64), jnp.float32),
            jax.ShapeDtypeStruct((ESF, 32), jnp.float32),
            jax.ShapeDtypeStruct((ESF, 32), jnp.float32),
        ),
        mesh=plsc.VectorSubcoreMesh(core_axis_name="c", subcore_axis_name="s",
                                    num_cores=NC, num_subcores=NS),
        scratch_types=[
            pltpu.VMEM((CH_T,), jnp.int32),
            pltpu.VMEM((CH_T, 64), jnp.float32),
            pltpu.VMEM((CH_F,), jnp.int32),
            pltpu.VMEM((CH_F, 32), jnp.float32),
            pltpu.SemaphoreType.DMA,
        ],
        compiler_params=pltpu.CompilerParams(use_tc_tiling_on_sc=False),
    )


def _mlp_t_body(gf_ref, gp_ref, ea_ref,
                cff_ref, bff1_ref, wff2_ref, bff2_ref,
                cfb_ref, bfb1_ref, wfb2_ref, bfb2_ref,
                fff_ref, ffb_ref):
    gf = gf_ref[...]
    gp = gp_ref[...]
    ea = ea_ref[...]
    qff = jnp.dot(ea, cff_ref[...], preferred_element_type=jnp.float32) + bff1_ref[...]
    h = jnp.maximum(gf[:, 0:32] + gp[:, 0:32] + qff, 0.0)
    fff_ref[...] = jnp.maximum(
        jnp.dot(h, wff2_ref[...], preferred_element_type=jnp.float32) + bff2_ref[...], 0.0)
    qfb = jnp.dot(ea, cfb_ref[...], preferred_element_type=jnp.float32) + bfb1_ref[...]
    h2 = jnp.maximum(gf[:, 32:64] + gp[:, 32:64] + qfb, 0.0)
    ffb_ref[...] = jnp.maximum(
        jnp.dot(h2, wfb2_ref[...], preferred_element_type=jnp.float32) + bfb2_ref[...], 0.0)


def _mlp_t(g_fut, g_past, edge_attr, cff, bff1, wff2, bff2, cfb, bfb1, wfb2, bfb2):
    be = 4000
    wspec = lambda shape: pl.BlockSpec(shape, lambda i: (0, 0))
    return pl.pallas_call(
        _mlp_t_body,
        grid=(E // be,),
        in_specs=[
            pl.BlockSpec((be, 64), lambda i: (i, 0)),
            pl.BlockSpec((be, 64), lambda i: (i, 0)),
            pl.BlockSpec((be, 16), lambda i: (i, 0)),
            wspec((16, H1)), wspec((1, H1)), wspec((H1, H2)), wspec((1, H2)),
            wspec((16, H1)), wspec((1, H1)), wspec((H1, H2)), wspec((1, H2)),
        ],
        out_specs=[
            pl.BlockSpec((be, 16), lambda i: (i, 0)),
            pl.BlockSpec((be, 16), lambda i: (i, 0)),
        ],
        out_shape=[
            jax.ShapeDtypeStruct((E, 16), jnp.float32),
            jax.ShapeDtypeStruct((E, 16), jnp.float32),
        ],
    )(g_fut, g_past, edge_attr, cff, bff1, wff2, bff2, cfb, bfb1, wfb2, bfb2)


def _mlp_f_body(ge_ref, gl_ref, ea_ref, cfr_ref, bfr1_ref, wfr2_ref, bfr2_ref, out_ref):
    q = jnp.dot(ea_ref[...], cfr_ref[...], preferred_element_type=jnp.float32) + bfr1_ref[...]
    h = jnp.maximum(ge_ref[...] + gl_ref[...] + q, 0.0)
    out_ref[...] = jnp.maximum(
        jnp.dot(h, wfr2_ref[...], preferred_element_type=jnp.float32) + bfr2_ref[...], 0.0)


def _mlp_f(g_early, g_later, attr, cfr, bfr1, wfr2, bfr2):
    be = 4000
    wspec = lambda shape: pl.BlockSpec(shape, lambda i: (0, 0))
    return pl.pallas_call(
        _mlp_f_body,
        grid=(ESF // be,),
        in_specs=[
            pl.BlockSpec((be, 32), lambda i: (i, 0)),
            pl.BlockSpec((be, 32), lambda i: (i, 0)),
            pl.BlockSpec((be, 16), lambda i: (i, 0)),
            wspec((16, H1)), wspec((1, H1)), wspec((H1, H2)), wspec((1, H2)),
        ],
        out_specs=pl.BlockSpec((be, 16), lambda i: (i, 0)),
        out_shape=jax.ShapeDtypeStruct((ESF, 16), jnp.float32),
    )(g_early, g_later, attr, cfr, bfr1, wfr2, bfr2)


def _sc_scatter_body(fff, ffb, ffr, fut, past, early, later, zrows, out,
                     acc_ff, acc_fr, acc_fb, idx_t, flow_t, idx_f, flow_f, sem):
    c = lax.axis_index("c")
    s = lax.axis_index("s")
    r0 = s * ROWS_PER_SUB

    pltpu.sync_copy(zrows.at[pl.ds(r0, ROWS_PER_SUB)], acc_ff.at[pl.ds(r0, ROWS_PER_SUB)])
    pltpu.sync_copy(zrows.at[pl.ds(r0, ROWS_PER_SUB)], acc_fr.at[pl.ds(r0, ROWS_PER_SUB)])
    pltpu.sync_copy(zrows.at[pl.ds(r0, ROWS_PER_SUB)], acc_fb.at[pl.ds(r0, ROWS_PER_SUB)])
    plsc.subcore_barrier()

    w = c * NS + s
    bt = w * EPW

    def t_step(i, _):
        b = bt + i * CH_T
        pltpu.sync_copy(fut.at[pl.ds(b, CH_T)], idx_t)
        pltpu.sync_copy(fff.at[pl.ds(b, CH_T)], flow_t)
        pltpu.sync_copy(flow_t, acc_ff.at[idx_t], add=True)
        pltpu.sync_copy(past.at[pl.ds(b, CH_T)], idx_t)
        pltpu.sync_copy(ffb.at[pl.ds(b, CH_T)], flow_t)
        pltpu.sync_copy(flow_t, acc_fb.at[idx_t], add=True)
        return 0

    lax.fori_loop(0, NT, t_step, 0)

    bf = w * FPW

    def f_step(i, _):
        b = bf + i * CH_F
        pltpu.sync_copy(ffr.at[pl.ds(b, CH_F)], flow_f)
        pltpu.sync_copy(early.at[pl.ds(b, CH_F)], idx_f)
        pltpu.sync_copy(flow_f, acc_fr.at[idx_f], add=True)
        pltpu.sync_copy(later.at[pl.ds(b, CH_F)], idx_f)
        pltpu.sync_copy(flow_f, acc_fr.at[idx_f], add=True)
        return 0

    lax.fori_loop(0, NF, f_step, 0)
    plsc.subcore_barrier()

    pltpu.sync_copy(acc_ff.at[pl.ds(r0, ROWS_PER_SUB)], out.at[c, 0, pl.ds(r0, ROWS_PER_SUB)])
    pltpu.sync_copy(acc_fr.at[pl.ds(r0, ROWS_PER_SUB)], out.at[c, 1, pl.ds(r0, ROWS_PER_SUB)])
    pltpu.sync_copy(acc_fb.at[pl.ds(r0, ROWS_PER_SUB)], out.at[c, 2, pl.ds(r0, ROWS_PER_SUB)])


@functools.cache
def _sc_scatter():
    return pl.kernel(
        _sc_scatter_body,
        out_type=jax.ShapeDtypeStruct((NC, 3, N, 16), jnp.float32),
        mesh=plsc.VectorSubcoreMesh(core_axis_name="c", subcore_axis_name="s",
                                    num_cores=NC, num_subcores=NS),
        scratch_types=[
            pltpu.VMEM_SHARED((N, 16), jnp.float32),
            pltpu.VMEM_SHARED((N, 16), jnp.float32),
            pltpu.VMEM_SHARED((N, 16), jnp.float32),
            pltpu.VMEM((CH_T,), jnp.int32),
            pltpu.VMEM((CH_T, 16), jnp.float32),
            pltpu.VMEM((CH_F,), jnp.int32),
            pltpu.VMEM((CH_F, 16), jnp.float32),
            pltpu.SemaphoreType.DMA,
        ],
        compiler_params=pltpu.CompilerParams(use_tc_tiling_on_sc=False),
    )


def _final_body(p_ref, wt_ref, bt_ref, out_ref):
    p = p_ref[...]
    tot = p[0] + p[1]
    ft = jnp.concatenate([tot[0], tot[1], tot[2]], axis=1)
    out_ref[...] = jnp.maximum(
        jnp.dot(ft, wt_ref[...], preferred_element_type=jnp.float32) + bt_ref[...], 0.0)


def _final(partials, wt1, bt1):
    bn = 2000
    return pl.pallas_call(
        _final_body,
        grid=(N // bn,),
        in_specs=[
            pl.BlockSpec((NC, 3, bn, 16), lambda i: (0, 0, i, 0)),
            pl.BlockSpec((48, 128), lambda i: (0, 0)),
            pl.BlockSpec((1, 128), lambda i: (0, 0)),
        ],
        out_specs=pl.BlockSpec((bn, 128), lambda i: (i, 0)),
        out_shape=jax.ShapeDtypeStruct((N, 128), jnp.float32),
    )(partials, wt1, bt1)


def kernel(x, edge_index, edge_attr, same_frame_edge_index, same_frame_edge_attr,
           Wff1, bff1, Wff2, bff2,
           Wfb1, bfb1, Wfb2, bfb2,
           Wfr1, bfr1, Wfr2, bfr2,
           Wt1, bt1):
    past = edge_index[0]
    fut = edge_index[1]
    early = same_frame_edge_index[0]
    later = same_frame_edge_index[1]

    wcat = jnp.concatenate([
        Wff1[:D], Wfb1[D:2 * D],
        Wff1[D:2 * D], Wfb1[:D],
        Wfr1[:D], Wfr1[D:2 * D],
    ], axis=1)

    tf_, tp_, te_, tl_ = _proj(x, wcat)
    g_fut, g_past, g_early, g_later = _sc_gather()(tf_, tp_, te_, tl_, fut, past, early, later)

    fff, ffb = _mlp_t(g_fut, g_past, edge_attr,
                      Wff1[2 * D:], bff1[None], Wff2, bff2[None],
                      Wfb1[2 * D:], bfb1[None], Wfb2, bfb2[None])
    ffr = _mlp_f(g_early, g_later, same_frame_edge_attr,
                 Wfr1[2 * D:], bfr1[None], Wfr2, bfr2[None])

    zrows = jnp.zeros((N, 16), jnp.float32)
    partials = _sc_scatter()(fff, ffb, ffr, fut, past, early, later, zrows)

    return _final(partials, Wt1, bt1[None])


# speedup vs baseline: 2.6818x; 1.1777x over previous
import functools

import jax
import jax.numpy as jnp
from jax import lax
from jax.experimental import pallas as pl
from jax.experimental.pallas import tpu as pltpu
from jax.experimental.pallas import tpu_sc as plsc

N = 10000
E = 320000
ESF = 160000
D = 128
H1 = 32
H2 = 16

NC = 2
NS = 16
NW = NC * NS

EPW = E // NW
FPW = ESF // NW
CH_T = 80
CH_F = 40
NT = EPW // CH_T
NF = FPW // CH_F

SPW = 2 * E // NW
NSC = SPW // CH_T

BE_T = 6400
BE_F = 3200

ACC_ROWS = 3 * N
ZPW = ACC_ROWS // NS


def _proj_body(x_ref, w_ref, tf_ref, tp_ref, te_ref, tl_ref):
    p = jnp.dot(x_ref[...], w_ref[...], preferred_element_type=jnp.float32)
    tf_ref[...] = p[:, 0:64]
    tp_ref[...] = p[:, 64:128]
    te_ref[...] = p[:, 128:160]
    tl_ref[...] = p[:, 160:192]


def _proj(x, wcat):
    return pl.pallas_call(
        _proj_body,
        out_shape=[
            jax.ShapeDtypeStruct((N, 64), jnp.float32),
            jax.ShapeDtypeStruct((N, 64), jnp.float32),
            jax.ShapeDtypeStruct((N, 32), jnp.float32),
            jax.ShapeDtypeStruct((N, 32), jnp.float32),
        ],
    )(x, wcat)


def _sc_gather_body(tf, tp, te, tl, fut, past, early, later,
                    g_fut, g_past, g_early, g_later,
                    idx_t, rows_t, idx_f, rows_f, sem):
    wid = lax.axis_index("s") * NC + lax.axis_index("c")

    bt = wid * EPW

    def t_step(i, _):
        b = bt + i * CH_T
        pltpu.sync_copy(fut.at[pl.ds(b, CH_T)], idx_t)
        pltpu.async_copy(tf.at[idx_t], rows_t, sem).wait()
        pltpu.sync_copy(rows_t, g_fut.at[pl.ds(b, CH_T)])
        pltpu.sync_copy(past.at[pl.ds(b, CH_T)], idx_t)
        pltpu.async_copy(tp.at[idx_t], rows_t, sem).wait()
        pltpu.sync_copy(rows_t, g_past.at[pl.ds(b, CH_T)])
        return 0

    lax.fori_loop(0, NT, t_step, 0)

    bf = wid * FPW

    def f_step(i, _):
        b = bf + i * CH_F
        pltpu.sync_copy(early.at[pl.ds(b, CH_F)], idx_f)
        pltpu.async_copy(te.at[idx_f], rows_f, sem).wait()
        pltpu.sync_copy(rows_f, g_early.at[pl.ds(b, CH_F)])
        pltpu.sync_copy(later.at[pl.ds(b, CH_F)], idx_f)
        pltpu.async_copy(tl.at[idx_f], rows_f, sem).wait()
        pltpu.sync_copy(rows_f, g_later.at[pl.ds(b, CH_F)])
        return 0

    lax.fori_loop(0, NF, f_step, 0)


@functools.cache
def _sc_gather():
    return pl.kernel(
        _sc_gather_body,
        out_type=(
            jax.ShapeDtypeStruct((E, 64), jnp.float32),
            jax.ShapeDtypeStruct((E, 64), jnp.float32),
            jax.ShapeDtypeStruct((ESF, 32), jnp.float32),
            jax.ShapeDtypeStruct((ESF, 32), jnp.float32),
        ),
        mesh=plsc.VectorSubcoreMesh(core_axis_name="c", subcore_axis_name="s",
                                    num_cores=NC, num_subcores=NS),
        scratch_types=[
            pltpu.VMEM((CH_T,), jnp.int32),
            pltpu.VMEM((CH_T, 64), jnp.float32),
            pltpu.VMEM((CH_F,), jnp.int32),
            pltpu.VMEM((CH_F, 32), jnp.float32),
            pltpu.SemaphoreType.DMA,
        ],
        compiler_params=pltpu.CompilerParams(use_tc_tiling_on_sc=False),
    )


def _mlp_t_body(gf_ref, gp_ref, a8_ref, w8_ref, b1_ref, w2d_ref, b2_ref, out_ref):
    q8 = jnp.dot(a8_ref[...], w8_ref[...], preferred_element_type=jnp.float32)
    q2 = q8.reshape(BE_T // 2, 128)
    h = jnp.maximum(gf_ref[...] + gp_ref[...] + q2 + b1_ref[...], 0.0)
    m = BE_T // 4
    lo = jnp.dot(h[0:m], w2d_ref[...], preferred_element_type=jnp.float32)
    hi = jnp.dot(h[m:2 * m], w2d_ref[...], preferred_element_type=jnp.float32)
    out_ref[...] = jnp.maximum(jnp.concatenate([lo, hi], axis=1) + b2_ref[...], 0.0)


def _mlp_t(g2f, g2p, attr8, w8, b1t, w2d, b2t):
    be = BE_T
    wspec = lambda shape: pl.BlockSpec(shape, lambda i: (0, 0))
    return pl.pallas_call(
        _mlp_t_body,
        grid=(E // be,),
        in_specs=[
            pl.BlockSpec((be // 2, 128), lambda i: (i, 0)),
            pl.BlockSpec((be // 2, 128), lambda i: (i, 0)),
            pl.BlockSpec((be // 8, 128), lambda i: (i, 0)),
            wspec((128, 512)), wspec((1, 128)), wspec((128, 64)), wspec((1, 128)),
        ],
        out_specs=pl.BlockSpec((be // 4, 128), lambda i: (i, 0)),
        out_shape=jax.ShapeDtypeStruct((E // 4, 128), jnp.float32),
    )(g2f, g2p, attr8, w8, b1t, w2d, b2t)


def _mlp_f_body(ge_ref, gl_ref, a8_ref, w8_ref, b1_ref, w2d_ref, b2_ref, out_ref):
    q8 = jnp.dot(a8_ref[...], w8_ref[...], preferred_element_type=jnp.float32)
    q4 = q8.reshape(BE_F // 4, 128)
    h = jnp.maximum(ge_ref[...] + gl_ref[...] + q4 + b1_ref[...], 0.0)
    m = BE_F // 8
    lo = jnp.dot(h[0:m], w2d_ref[...], preferred_element_type=jnp.float32)
    hi = jnp.dot(h[m:2 * m], w2d_ref[...], preferred_element_type=jnp.float32)
    out_ref[...] = jnp.maximum(jnp.concatenate([lo, hi], axis=1) + b2_ref[...], 0.0)


def _mlp_f(g4e, g4l, attr8, w8, b1t, w2d, b2t):
    be = BE_F
    wspec = lambda shape: pl.BlockSpec(shape, lambda i: (0, 0))
    return pl.pallas_call(
        _mlp_f_body,
        grid=(ESF // be,),
        in_specs=[
            pl.BlockSpec((be // 4, 128), lambda i: (i, 0)),
            pl.BlockSpec((be // 4, 128), lambda i: (i, 0)),
            pl.BlockSpec((be // 8, 128), lambda i: (i, 0)),
            wspec((128, 256)), wspec((1, 128)), wspec((128, 64)), wspec((1, 128)),
        ],
        out_specs=pl.BlockSpec((be // 8, 128), lambda i: (i, 0)),
        out_shape=jax.ShapeDtypeStruct((ESF // 8, 128), jnp.float32),
    )(g4e, g4l, attr8, w8, b1t, w2d, b2t)


def _sc_scatter_body(fl_t, fl_f, idx_comb, idx_e2, idx_l2, zrows, out,
                     acc, idx_t, flow_t, idx_f, flow_f, sem):
    c = lax.axis_index("c")
    s = lax.axis_index("s")
    r0 = s * ZPW

    pltpu.sync_copy(zrows.at[pl.ds(r0, ZPW)], acc.at[pl.ds(r0, ZPW)])
    plsc.subcore_barrier()

    w = c * NS + s
    bt = w * SPW

    def t_step(i, _):
        b = bt + i * CH_T
        pltpu.sync_copy(idx_comb.at[pl.ds(b, CH_T)], idx_t)
        pltpu.sync_copy(fl_t.at[pl.ds(b, CH_T)], flow_t)
        pltpu.sync_copy(flow_t, acc.at[idx_t], add=True)
        return 0

    lax.fori_loop(0, NSC, t_step, 0)

    bf = w * FPW

    def f_step(i, _):
        b = bf + i * CH_F
        pltpu.sync_copy(fl_f.at[pl.ds(b, CH_F)], flow_f)
        pltpu.sync_copy(idx_e2.at[pl.ds(b, CH_F)], idx_f)
        pltpu.sync_copy(flow_f, acc.at[idx_f], add=True)
        pltpu.sync_copy(idx_l2.at[pl.ds(b, CH_F)], idx_f)
        pltpu.sync_copy(flow_f, acc.at[idx_f], add=True)
        return 0

    lax.fori_loop(0, NF, f_step, 0)
    plsc.subcore_barrier()

    pltpu.sync_copy(acc.at[pl.ds(r0, ZPW)], out.at[c, pl.ds(r0, ZPW)])


@functools.cache
def _sc_scatter():
    return pl.kernel(
        _sc_scatter_body,
        out_type=jax.ShapeDtypeStruct((NC, ACC_ROWS, 16), jnp.float32),
        mesh=plsc.VectorSubcoreMesh(core_axis_name="c", subcore_axis_name="s",
                                    num_cores=NC, num_subcores=NS),
        scratch_types=[
            pltpu.VMEM_SHARED((ACC_ROWS, 16), jnp.float32),
            pltpu.VMEM((CH_T,), jnp.int32),
            pltpu.VMEM((CH_T, 16), jnp.float32),
            pltpu.VMEM((CH_F,), jnp.int32),
            pltpu.VMEM((CH_F, 16), jnp.float32),
            pltpu.SemaphoreType.DMA,
        ],
        compiler_params=pltpu.CompilerParams(use_tc_tiling_on_sc=False),
    )


def _final_body(p_ref, k0_ref, k1_ref, k2_ref, bt_ref, out_ref):
    r = N // 8
    tot = p_ref[0] + p_ref[1]
    acc = jnp.dot(tot[0:r], k0_ref[...], preferred_element_type=jnp.float32)
    acc += jnp.dot(tot[r:2 * r], k1_ref[...], preferred_element_type=jnp.float32)
    acc += jnp.dot(tot[2 * r:3 * r], k2_ref[...], preferred_element_type=jnp.float32)
    out_ref[...] = jnp.maximum(acc + bt_ref[...], 0.0)


def _final(partials, k0, k1, k2, btile):
    return pl.pallas_call(
        _final_body,
        out_shape=jax.ShapeDtypeStruct((N // 8, 1024), jnp.float32),
    )(partials, k0, k1, k2, btile)


def kernel(x, edge_index, edge_attr, same_frame_edge_index, same_frame_edge_attr,
           Wff1, bff1, Wff2, bff2,
           Wfb1, bfb1, Wfb2, bfb2,
           Wfr1, bfr1, Wfr2, bfr2,
           Wt1, bt1):
    f32 = jnp.float32
    past = edge_index[0]
    fut = edge_index[1]
    early = same_frame_edge_index[0]
    later = same_frame_edge_index[1]

    wcat = jnp.concatenate([
        Wff1[:D], Wfb1[D:2 * D],
        Wff1[D:2 * D], Wfb1[:D],
        Wfr1[:D], Wfr1[D:2 * D],
    ], axis=1)

    tf_, tp_, te_, tl_ = _proj(x, wcat)
    g_fut, g_past, g_early, g_later = _sc_gather()(
        tf_, tp_, te_, tl_, fut, past, early, later)

    eye8 = jnp.eye(8, dtype=f32)
    cboth = jnp.concatenate([Wff1[2 * D:], Wfb1[2 * D:]], axis=1)
    w8_t = jnp.kron(eye8, cboth)
    b1_t = jnp.tile(jnp.concatenate([bff1, bfb1]), 2)[None]
    w2d_t = jnp.kron(jnp.eye(2, dtype=f32),
                     jnp.concatenate([
                         jnp.concatenate([Wff2, jnp.zeros((H1, H2), f32)], axis=1),
                         jnp.concatenate([jnp.zeros((H1, H2), f32), Wfb2], axis=1),
                     ], axis=0))
    b2_t = jnp.tile(jnp.concatenate([bff2, bfb2]), 4)[None]

    fl_t = _mlp_t(g_fut.reshape(E // 2, 128), g_past.reshape(E // 2, 128),
                  edge_attr.reshape(E // 8, 128), w8_t, b1_t, w2d_t, b2_t)

    w8_f = jnp.kron(eye8, Wfr1[2 * D:])
    b1_f = jnp.tile(bfr1, 4)[None]
    w2d_f = jnp.kron(jnp.eye(4, dtype=f32), Wfr2)
    b2_f = jnp.tile(bfr2, 8)[None]

    fl_f = _mlp_f(g_early.reshape(ESF // 4, 128), g_later.reshape(ESF // 4, 128),
                  same_frame_edge_attr.reshape(ESF // 8, 128),
                  w8_f, b1_f, w2d_f, b2_f)

    ids_t = jnp.stack([fut, past + N], axis=1)
    idx_comb = (ids_t.reshape(E // BE_T, 2, BE_T // 4, 2, 2)
                .transpose(0, 2, 1, 3, 4).reshape(2 * E))
    perm_f = lambda v: (v.reshape(ESF // BE_F, 2, BE_F // 8, 4)
                        .transpose(0, 2, 1, 3).reshape(ESF))
    idx_e2 = perm_f(early + 2 * N)
    idx_l2 = perm_f(later + 2 * N)
    zrows = jnp.zeros((ACC_ROWS, 16), f32)

    partials = _sc_scatter()(fl_t.reshape(2 * E, 16), fl_f.reshape(ESF, 16),
                             idx_comb, idx_e2, idx_l2, zrows)

    k0 = jnp.kron(eye8, Wt1[0:16])
    k1 = jnp.kron(eye8, Wt1[32:48])
    k2 = jnp.kron(eye8, Wt1[16:32])
    btile = jnp.tile(bt1, 8)[None]

    out_p = _final(partials.reshape(NC, 3 * N // 8, 128), k0, k1, k2, btile)
    return out_p.reshape(N, 128)


# speedup vs baseline: 3.2179x; 1.1999x over previous
import functools

import jax
import jax.numpy as jnp
from jax import lax
from jax.experimental import pallas as pl
from jax.experimental.pallas import tpu as pltpu
from jax.experimental.pallas import tpu_sc as plsc

N = 10000
E = 320000
ESF = 160000
D = 128
H1 = 32
H2 = 16

NC = 2
NS = 16
NW = NC * NS

EPW = E // NW
FPW = ESF // NW
CH_T = 80
CH_F = 40
NT = EPW // CH_T
NF = FPW // CH_F

SPW = 2 * E // NW
NSC = SPW // CH_T

BE_T = 6400
BE_F = 3200

ACC_ROWS = 3 * N
ZPW = ACC_ROWS // NS


def _proj_body(x_ref, w_ref, tf_ref, tp_ref, te_ref, tl_ref):
    p = jnp.dot(x_ref[...], w_ref[...], preferred_element_type=jnp.float32)
    tf_ref[...] = p[:, 0:64]
    tp_ref[...] = p[:, 64:128]
    te_ref[...] = p[:, 128:160]
    tl_ref[...] = p[:, 160:192]


def _proj(x, wcat):
    return pl.pallas_call(
        _proj_body,
        out_shape=[
            jax.ShapeDtypeStruct((N, 64), jnp.float32),
            jax.ShapeDtypeStruct((N, 64), jnp.float32),
            jax.ShapeDtypeStruct((N, 32), jnp.float32),
            jax.ShapeDtypeStruct((N, 32), jnp.float32),
        ],
    )(x, wcat)


def _pair_loop(n_chunks, do_pair, do_single, drain, prime):
    prime()

    def body(k, _):
        do_pair(k)
        return 0

    lax.fori_loop(0, n_chunks // 2, body, 0)
    drain()
    if n_chunks % 2:
        do_single(n_chunks - 1)


def _sc_gather_body(tf, tp, te, tl, fut2, past2, early2, later2,
                    g_fut, g_past, g_early, g_later,
                    idxf_v, idxp_v, idxe_v, idxl_v,
                    rf0, rf1, rp0, rp1, re0, re1, rl0, rl1,
                    semg0, semg1, sems):
    wid = lax.axis_index("s") * NC + lax.axis_index("c")
    bt = wid * EPW
    bf = wid * FPW

    pltpu.sync_copy(fut2.at[pl.ds(wid * NT, NT)], idxf_v)
    pltpu.sync_copy(past2.at[pl.ds(wid * NT, NT)], idxp_v)
    pltpu.sync_copy(early2.at[pl.ds(wid * NF, NF)], idxe_v)
    pltpu.sync_copy(later2.at[pl.ds(wid * NF, NF)], idxl_v)

    def t_drain(b):
        pltpu.make_async_copy(rf0, g_fut.at[pl.ds(b, CH_T)], sems).wait()
        pltpu.make_async_copy(rp0, g_past.at[pl.ds(b, CH_T)], sems).wait()
        pltpu.make_async_copy(rf1, g_fut.at[pl.ds(b, CH_T)], sems).wait()
        pltpu.make_async_copy(rp1, g_past.at[pl.ds(b, CH_T)], sems).wait()

    def t_pair(k):
        r0 = 2 * k
        b0 = bt + r0 * CH_T
        b1 = b0 + CH_T

        @pl.when(k > 0)
        def _():
            t_drain(b0)

        d1 = pltpu.async_copy(tf.at[idxf_v.at[r0]], rf0, semg0)
        d2 = pltpu.async_copy(tp.at[idxp_v.at[r0]], rp0, semg0)
        d3 = pltpu.async_copy(tf.at[idxf_v.at[r0 + 1]], rf1, semg1)
        d4 = pltpu.async_copy(tp.at[idxp_v.at[r0 + 1]], rp1, semg1)
        d1.wait()
        d2.wait()
        pltpu.async_copy(rf0, g_fut.at[pl.ds(b0, CH_T)], sems)
        pltpu.async_copy(rp0, g_past.at[pl.ds(b0, CH_T)], sems)
        d3.wait()
        d4.wait()
        pltpu.async_copy(rf1, g_fut.at[pl.ds(b1, CH_T)], sems)
        pltpu.async_copy(rp1, g_past.at[pl.ds(b1, CH_T)], sems)

    def t_single(i):
        b = bt + i * CH_T
        pltpu.async_copy(tf.at[idxf_v.at[i]], rf0, semg0).wait()
        pltpu.async_copy(rf0, g_fut.at[pl.ds(b, CH_T)], sems).wait()
        pltpu.async_copy(tp.at[idxp_v.at[i]], rp0, semg0).wait()
        pltpu.async_copy(rp0, g_past.at[pl.ds(b, CH_T)], sems).wait()

    _pair_loop(NT, t_pair, t_single, lambda: t_drain(bt), lambda: None)

    def f_drain(b):
        pltpu.make_async_copy(re0, g_early.at[pl.ds(b, CH_F)], sems).wait()
        pltpu.make_async_copy(rl0, g_later.at[pl.ds(b, CH_F)], sems).wait()
        pltpu.make_async_copy(re1, g_early.at[pl.ds(b, CH_F)], sems).wait()
        pltpu.make_async_copy(rl1, g_later.at[pl.ds(b, CH_F)], sems).wait()

    def f_pair(k):
        r0 = 2 * k
        b0 = bf + r0 * CH_F
        b1 = b0 + CH_F

        @pl.when(k > 0)
        def _():
            f_drain(b0)

        d1 = pltpu.async_copy(te.at[idxe_v.at[r0]], re0, semg0)
        d2 = pltpu.async_copy(tl.at[idxl_v.at[r0]], rl0, semg0)
        d3 = pltpu.async_copy(te.at[idxe_v.at[r0 + 1]], re1, semg1)
        d4 = pltpu.async_copy(tl.at[idxl_v.at[r0 + 1]], rl1, semg1)
        d1.wait()
        d2.wait()
        pltpu.async_copy(re0, g_early.at[pl.ds(b0, CH_F)], sems)
        pltpu.async_copy(rl0, g_later.at[pl.ds(b0, CH_F)], sems)
        d3.wait()
        d4.wait()
        pltpu.async_copy(re1, g_early.at[pl.ds(b1, CH_F)], sems)
        pltpu.async_copy(rl1, g_later.at[pl.ds(b1, CH_F)], sems)

    def f_single(i):
        b = bf + i * CH_F
        pltpu.async_copy(te.at[idxe_v.at[i]], re0, semg0).wait()
        pltpu.async_copy(re0, g_early.at[pl.ds(b, CH_F)], sems).wait()
        pltpu.async_copy(tl.at[idxl_v.at[i]], rl0, semg0).wait()
        pltpu.async_copy(rl0, g_later.at[pl.ds(b, CH_F)], sems).wait()

    _pair_loop(NF, f_pair, f_single, lambda: f_drain(bf), lambda: None)


@functools.cache
def _sc_gather():
    return pl.kernel(
        _sc_gather_body,
        out_type=(
            jax.ShapeDtypeStruct((E, 64), jnp.float32),
            jax.ShapeDtypeStruct((E, 64), jnp.float32),
            jax.ShapeDtypeStruct((ESF, 32), jnp.float32),
            jax.ShapeDtypeStruct((ESF, 32), jnp.float32),
        ),
        mesh=plsc.VectorSubcoreMesh(core_axis_name="c", subcore_axis_name="s",
                                    num_cores=NC, num_subcores=NS),
        scratch_types=[
            pltpu.VMEM((NT, CH_T), jnp.int32),
            pltpu.VMEM((NT, CH_T), jnp.int32),
            pltpu.VMEM((NF, CH_F), jnp.int32),
            pltpu.VMEM((NF, CH_F), jnp.int32),
            pltpu.VMEM((CH_T, 64), jnp.float32),
            pltpu.VMEM((CH_T, 64), jnp.float32),
            pltpu.VMEM((CH_T, 64), jnp.float32),
            pltpu.VMEM((CH_T, 64), jnp.float32),
            pltpu.VMEM((CH_F, 32), jnp.float32),
            pltpu.VMEM((CH_F, 32), jnp.float32),
            pltpu.VMEM((CH_F, 32), jnp.float32),
            pltpu.VMEM((CH_F, 32), jnp.float32),
            pltpu.SemaphoreType.DMA,
            pltpu.SemaphoreType.DMA,
            pltpu.SemaphoreType.DMA,
        ],
        compiler_params=pltpu.CompilerParams(use_tc_tiling_on_sc=False),
    )


def _mlp_t_body(gf_ref, gp_ref, a8_ref, w8_ref, b1_ref, w2d_ref, b2_ref, out_ref):
    q8 = jnp.dot(a8_ref[...], w8_ref[...], preferred_element_type=jnp.float32)
    q2 = q8.reshape(BE_T // 2, 128)
    h = jnp.maximum(gf_ref[...] + gp_ref[...] + q2 + b1_ref[...], 0.0)
    m = BE_T // 4
    lo = jnp.dot(h[0:m], w2d_ref[...], preferred_element_type=jnp.float32)
    hi = jnp.dot(h[m:2 * m], w2d_ref[...], preferred_element_type=jnp.float32)
    out_ref[...] = jnp.maximum(jnp.concatenate([lo, hi], axis=1) + b2_ref[...], 0.0)


def _mlp_t(g2f, g2p, attr8, w8, b1t, w2d, b2t):
    be = BE_T
    wspec = lambda shape: pl.BlockSpec(shape, lambda i: (0, 0))
    return pl.pallas_call(
        _mlp_t_body,
        grid=(E // be,),
        in_specs=[
            pl.BlockSpec((be // 2, 128), lambda i: (i, 0)),
            pl.BlockSpec((be // 2, 128), lambda i: (i, 0)),
            pl.BlockSpec((be // 8, 128), lambda i: (i, 0)),
            wspec((128, 512)), wspec((1, 128)), wspec((128, 64)), wspec((1, 128)),
        ],
        out_specs=pl.BlockSpec((be // 4, 128), lambda i: (i, 0)),
        out_shape=jax.ShapeDtypeStruct((E // 4, 128), jnp.float32),
    )(g2f, g2p, attr8, w8, b1t, w2d, b2t)


def _mlp_f_body(ge_ref, gl_ref, a8_ref, w8_ref, b1_ref, w2d_ref, b2_ref, out_ref):
    q8 = jnp.dot(a8_ref[...], w8_ref[...], preferred_element_type=jnp.float32)
    q4 = q8.reshape(BE_F // 4, 128)
    h = jnp.maximum(ge_ref[...] + gl_ref[...] + q4 + b1_ref[...], 0.0)
    m = BE_F // 8
    lo = jnp.dot(h[0:m], w2d_ref[...], preferred_element_type=jnp.float32)
    hi = jnp.dot(h[m:2 * m], w2d_ref[...], preferred_element_type=jnp.float32)
    out_ref[...] = jnp.maximum(jnp.concatenate([lo, hi], axis=1) + b2_ref[...], 0.0)


def _mlp_f(g4e, g4l, attr8, w8, b1t, w2d, b2t):
    be = BE_F
    wspec = lambda shape: pl.BlockSpec(shape, lambda i: (0, 0))
    return pl.pallas_call(
        _mlp_f_body,
        grid=(ESF // be,),
        in_specs=[
            pl.BlockSpec((be // 4, 128), lambda i: (i, 0)),
            pl.BlockSpec((be // 4, 128), lambda i: (i, 0)),
            pl.BlockSpec((be // 8, 128), lambda i: (i, 0)),
            wspec((128, 256)), wspec((1, 128)), wspec((128, 64)), wspec((1, 128)),
        ],
        out_specs=pl.BlockSpec((be // 8, 128), lambda i: (i, 0)),
        out_shape=jax.ShapeDtypeStruct((ESF // 8, 128), jnp.float32),
    )(g4e, g4l, attr8, w8, b1t, w2d, b2t)


def _sc_scatter_body(fl_t, fl_f, idxc2, idxe2, idxl2, zrows, out,
                     acc, idxc_v, idxe_v, idxl_v, fb0, fb1, ff0, ff1,
                     sem0, sem1, sem2, sem3):
    c = lax.axis_index("c")
    s = lax.axis_index("s")
    r0 = s * ZPW

    pltpu.sync_copy(zrows.at[pl.ds(r0, ZPW)], acc.at[pl.ds(r0, ZPW)])

    w = c * NS + s
    bt = w * SPW
    bf = w * FPW

    pltpu.sync_copy(idxc2.at[pl.ds(w * NSC, NSC)], idxc_v)
    pltpu.sync_copy(idxe2.at[pl.ds(w * NF, NF)], idxe_v)
    pltpu.sync_copy(idxl2.at[pl.ds(w * NF, NF)], idxl_v)
    plsc.subcore_barrier()

    pltpu.async_copy(fl_t.at[pl.ds(bt, CH_T)], fb0, sem0)
    pltpu.async_copy(fl_t.at[pl.ds(bt + CH_T, CH_T)], fb1, sem1)

    def t_pair(k, _):
        i0 = 2 * k
        b0 = bt + i0 * CH_T
        pltpu.make_async_copy(fl_t.at[pl.ds(b0, CH_T)], fb0, sem0).wait()
        pltpu.sync_copy(fb0, acc.at[idxc_v.at[i0]], add=True)

        @pl.when(k + 1 < NSC // 2)
        def _():
            pltpu.async_copy(fl_t.at[pl.ds(b0 + 2 * CH_T, CH_T)], fb0, sem0)

        pltpu.make_async_copy(fl_t.at[pl.ds(b0, CH_T)], fb1, sem1).wait()
        pltpu.sync_copy(fb1, acc.at[idxc_v.at[i0 + 1]], add=True)

        @pl.when(k + 1 < NSC // 2)
        def _():
            pltpu.async_copy(fl_t.at[pl.ds(b0 + 3 * CH_T, CH_T)], fb1, sem1)

        return 0

    lax.fori_loop(0, NSC // 2, t_pair, 0)

    pltpu.async_copy(fl_f.at[pl.ds(bf, CH_F)], ff0, sem2)
    pltpu.async_copy(fl_f.at[pl.ds(bf + CH_F, CH_F)], ff1, sem3)

    def f_pair(k, _):
        i0 = 2 * k
        b0 = bf + i0 * CH_F
        pltpu.make_async_copy(fl_f.at[pl.ds(b0, CH_F)], ff0, sem2).wait()
        pltpu.sync_copy(ff0, acc.at[idxe_v.at[i0]], add=True)
        pltpu.sync_copy(ff0, acc.at[idxl_v.at[i0]], add=True)

        @pl.when(k + 1 < NF // 2)
        def _():
            pltpu.async_copy(fl_f.at[pl.ds(b0 + 2 * CH_F, CH_F)], ff0, sem2)

        pltpu.make_async_copy(fl_f.at[pl.ds(b0, CH_F)], ff1, sem3).wait()
        pltpu.sync_copy(ff1, acc.at[idxe_v.at[i0 + 1]], add=True)
        pltpu.sync_copy(ff1, acc.at[idxl_v.at[i0 + 1]], add=True)

        @pl.when(k + 1 < NF // 2)
        def _():
            pltpu.async_copy(fl_f.at[pl.ds(b0 + 3 * CH_F, CH_F)], ff1, sem3)

        return 0

    lax.fori_loop(0, NF // 2, f_pair, 0)

    bl = bf + (NF - 1) * CH_F
    pltpu.async_copy(fl_f.at[pl.ds(bl, CH_F)], ff0, sem2).wait()
    pltpu.sync_copy(ff0, acc.at[idxe_v.at[NF - 1]], add=True)
    pltpu.sync_copy(ff0, acc.at[idxl_v.at[NF - 1]], add=True)

    plsc.subcore_barrier()
    pltpu.sync_copy(acc.at[pl.ds(r0, ZPW)], out.at[c, pl.ds(r0, ZPW)])


@functools.cache
def _sc_scatter():
    return pl.kernel(
        _sc_scatter_body,
        out_type=jax.ShapeDtypeStruct((NC, ACC_ROWS, 16), jnp.float32),
        mesh=plsc.VectorSubcoreMesh(core_axis_name="c", subcore_axis_name="s",
                                    num_cores=NC, num_subcores=NS),
        scratch_types=[
            pltpu.VMEM_SHARED((ACC_ROWS, 16), jnp.float32),
            pltpu.VMEM((NSC, CH_T), jnp.int32),
            pltpu.VMEM((NF, CH_F), jnp.int32),
            pltpu.VMEM((NF, CH_F), jnp.int32),
            pltpu.VMEM((CH_T, 16), jnp.float32),
            pltpu.VMEM((CH_T, 16), jnp.float32),
            pltpu.VMEM((CH_F, 16), jnp.float32),
            pltpu.VMEM((CH_F, 16), jnp.float32),
            pltpu.SemaphoreType.DMA,
            pltpu.SemaphoreType.DMA,
            pltpu.SemaphoreType.DMA,
            pltpu.SemaphoreType.DMA,
        ],
        compiler_params=pltpu.CompilerParams(use_tc_tiling_on_sc=False),
    )


def _final_body(p_ref, k0_ref, k1_ref, k2_ref, bt_ref, out_ref):
    r = N // 8
    tot = p_ref[0] + p_ref[1]
    acc = jnp.dot(tot[0:r], k0_ref[...], preferred_element_type=jnp.float32)
    acc += jnp.dot(tot[r:2 * r], k1_ref[...], preferred_element_type=jnp.float32)
    acc += jnp.dot(tot[2 * r:3 * r], k2_ref[...], preferred_element_type=jnp.float32)
    out_ref[...] = jnp.maximum(acc + bt_ref[...], 0.0)


def _final(partials, k0, k1, k2, btile):
    return pl.pallas_call(
        _final_body,
        out_shape=jax.ShapeDtypeStruct((N // 8, 1024), jnp.float32),
    )(partials, k0, k1, k2, btile)


def kernel(x, edge_index, edge_attr, same_frame_edge_index, same_frame_edge_attr,
           Wff1, bff1, Wff2, bff2,
           Wfb1, bfb1, Wfb2, bfb2,
           Wfr1, bfr1, Wfr2, bfr2,
           Wt1, bt1):
    f32 = jnp.float32
    past = edge_index[0]
    fut = edge_index[1]
    early = same_frame_edge_index[0]
    later = same_frame_edge_index[1]

    wcat = jnp.concatenate([
        Wff1[:D], Wfb1[D:2 * D],
        Wff1[D:2 * D], Wfb1[:D],
        Wfr1[:D], Wfr1[D:2 * D],
    ], axis=1)

    tf_, tp_, te_, tl_ = _proj(x, wcat)
    g_fut, g_past, g_early, g_later = _sc_gather()(
        tf_, tp_, te_, tl_,
        fut.reshape(E // CH_T, CH_T), past.reshape(E // CH_T, CH_T),
        early.reshape(ESF // CH_F, CH_F), later.reshape(ESF // CH_F, CH_F))

    eye8 = jnp.eye(8, dtype=f32)
    cboth = jnp.concatenate([Wff1[2 * D:], Wfb1[2 * D:]], axis=1)
    w8_t = jnp.kron(eye8, cboth)
    b1_t = jnp.tile(jnp.concatenate([bff1, bfb1]), 2)[None]
    w2d_t = jnp.kron(jnp.eye(2, dtype=f32),
                     jnp.concatenate([
                         jnp.concatenate([Wff2, jnp.zeros((H1, H2), f32)], axis=1),
                         jnp.concatenate([jnp.zeros((H1, H2), f32), Wfb2], axis=1),
                     ], axis=0))
    b2_t = jnp.tile(jnp.concatenate([bff2, bfb2]), 4)[None]

    fl_t = _mlp_t(g_fut.reshape(E // 2, 128), g_past.reshape(E // 2, 128),
                  edge_attr.reshape(E // 8, 128), w8_t, b1_t, w2d_t, b2_t)

    w8_f = jnp.kron(eye8, Wfr1[2 * D:])
    b1_f = jnp.tile(bfr1, 4)[None]
    w2d_f = jnp.kron(jnp.eye(4, dtype=f32), Wfr2)
    b2_f = jnp.tile(bfr2, 8)[None]

    fl_f = _mlp_f(g_early.reshape(ESF // 4, 128), g_later.reshape(ESF // 4, 128),
                  same_frame_edge_attr.reshape(ESF // 8, 128),
                  w8_f, b1_f, w2d_f, b2_f)

    ids_t = jnp.stack([fut, past + N], axis=1)
    idx_comb = (ids_t.reshape(E // BE_T, 2, BE_T // 4, 2, 2)
                .transpose(0, 2, 1, 3, 4).reshape(2 * E))
    perm_f = lambda v: (v.reshape(ESF // BE_F, 2, BE_F // 8, 4)
                        .transpose(0, 2, 1, 3).reshape(ESF))
    idx_e2 = perm_f(early + 2 * N)
    idx_l2 = perm_f(later + 2 * N)
    zrows = jnp.zeros((ACC_ROWS, 16), f32)

    partials = _sc_scatter()(fl_t.reshape(2 * E, 16), fl_f.reshape(ESF, 16),
                             idx_comb.reshape(2 * E // CH_T, CH_T),
                             idx_e2.reshape(ESF // CH_F, CH_F),
                             idx_l2.reshape(ESF // CH_F, CH_F), zrows)

    k0 = jnp.kron(eye8, Wt1[0:16])
    k1 = jnp.kron(eye8, Wt1[32:48])
    k2 = jnp.kron(eye8, Wt1[16:32])
    btile = jnp.tile(bt1, 8)[None]

    out_p = _final(partials.reshape(NC, 3 * N // 8, 128), k0, k1, k2, btile)
    return out_p.reshape(N, 128)


# speedup vs baseline: 7.2317x; 2.2474x over previous
import functools

import jax
import jax.numpy as jnp
import numpy as np
from jax import lax
from jax.experimental import pallas as pl
from jax.experimental.pallas import tpu as pltpu
from jax.experimental.pallas import tpu_sc as plsc

N = 10000
E = 320000
ESF = 160000
D = 128
H1 = 32
H2 = 16

NC = 2
NS = 16
NW = NC * NS

EPW = E // NW
FPW = ESF // NW
CH_T = 80
CH_F = 40
NT = EPW // CH_T
NF = FPW // CH_F

SPW = 2 * E // NW
NSC = SPW // CH_T

BE_T = 6400
BE_F = 3200

ACC_ROWS = 3 * N
ZPW = ACC_ROWS // NS


def _temporal_perm():
    r = np.arange(2 * E, dtype=np.int64)
    R, u = r // 8, r % 8
    half, v = u // 4, u % 4
    blk, j = R // (BE_T // 4), R % (BE_T // 4)
    e = blk * BE_T + half * (BE_T // 2) + 2 * j + v // 2
    isfb = v % 2
    return (isfb * E + e).astype(np.int32)


def _frame_perm():
    r = np.arange(ESF, dtype=np.int64)
    R, u = r // 8, r % 8
    half, v = u // 4, u % 4
    blk, j = R // (BE_F // 8), R % (BE_F // 8)
    e = blk * BE_F + half * (BE_F // 2) + 4 * j + v
    return e.astype(np.int32)


_T_PERM = _temporal_perm()
_F_PERM = _frame_perm()


def _proj_body(x_ref, w_ref, tf_ref, tp_ref, te_ref, tl_ref):
    p = jnp.dot(x_ref[...], w_ref[...], preferred_element_type=jnp.float32)
    tf_ref[...] = p[:, 0:64]
    tp_ref[...] = p[:, 64:128]
    te_ref[...] = p[:, 128:160]
    tl_ref[...] = p[:, 160:192]


def _proj(x, wcat):
    return pl.pallas_call(
        _proj_body,
        out_shape=[
            jax.ShapeDtypeStruct((N, 64), jnp.float32),
            jax.ShapeDtypeStruct((N, 64), jnp.float32),
            jax.ShapeDtypeStruct((N, 32), jnp.float32),
            jax.ShapeDtypeStruct((N, 32), jnp.float32),
        ],
    )(x, wcat)


def _pair_loop(n_chunks, do_pair, do_single, drain, prime):
    prime()

    def body(k, _):
        do_pair(k)
        return 0

    lax.fori_loop(0, n_chunks // 2, body, 0)
    drain()
    if n_chunks % 2:
        do_single(n_chunks - 1)


def _sc_gather_body(tf, tp, te, tl, fut2, past2, early2, later2,
                    g_fut, g_past, g_early, g_later,
                    idxf_v, idxp_v, idxe_v, idxl_v,
                    rf0, rf1, rp0, rp1, re0, re1, rl0, rl1,
                    semg0, semg1, sems):
    wid = lax.axis_index("s") * NC + lax.axis_index("c")
    bt = wid * EPW
    bf = wid * FPW

    pltpu.sync_copy(fut2.at[pl.ds(wid * NT, NT)], idxf_v)
    pltpu.sync_copy(past2.at[pl.ds(wid * NT, NT)], idxp_v)
    pltpu.sync_copy(early2.at[pl.ds(wid * NF, NF)], idxe_v)
    pltpu.sync_copy(later2.at[pl.ds(wid * NF, NF)], idxl_v)

    def t_drain(b):
        pltpu.make_async_copy(rf0, g_fut.at[pl.ds(b, CH_T)], sems).wait()
        pltpu.make_async_copy(rp0, g_past.at[pl.ds(b, CH_T)], sems).wait()
        pltpu.make_async_copy(rf1, g_fut.at[pl.ds(b, CH_T)], sems).wait()
        pltpu.make_async_copy(rp1, g_past.at[pl.ds(b, CH_T)], sems).wait()

    def t_pair(k):
        r0 = 2 * k
        b0 = bt + r0 * CH_T
        b1 = b0 + CH_T

        @pl.when(k > 0)
        def _():
            t_drain(b0)

        d1 = pltpu.async_copy(tf.at[idxf_v.at[r0]], rf0, semg0)
        d2 = pltpu.async_copy(tp.at[idxp_v.at[r0]], rp0, semg0)
        d3 = pltpu.async_copy(tf.at[idxf_v.at[r0 + 1]], rf1, semg1)
        d4 = pltpu.async_copy(tp.at[idxp_v.at[r0 + 1]], rp1, semg1)
        d1.wait()
        d2.wait()
        pltpu.async_copy(rf0, g_fut.at[pl.ds(b0, CH_T)], sems)
        pltpu.async_copy(rp0, g_past.at[pl.ds(b0, CH_T)], sems)
        d3.wait()
        d4.wait()
        pltpu.async_copy(rf1, g_fut.at[pl.ds(b1, CH_T)], sems)
        pltpu.async_copy(rp1, g_past.at[pl.ds(b1, CH_T)], sems)

    def t_single(i):
        b = bt + i * CH_T
        pltpu.async_copy(tf.at[idxf_v.at[i]], rf0, semg0).wait()
        pltpu.async_copy(rf0, g_fut.at[pl.ds(b, CH_T)], sems).wait()
        pltpu.async_copy(tp.at[idxp_v.at[i]], rp0, semg0).wait()
        pltpu.async_copy(rp0, g_past.at[pl.ds(b, CH_T)], sems).wait()

    _pair_loop(NT, t_pair, t_single, lambda: t_drain(bt), lambda: None)

    def f_drain(b):
        pltpu.make_async_copy(re0, g_early.at[pl.ds(b, CH_F)], sems).wait()
        pltpu.make_async_copy(rl0, g_later.at[pl.ds(b, CH_F)], sems).wait()
        pltpu.make_async_copy(re1, g_early.at[pl.ds(b, CH_F)], sems).wait()
        pltpu.make_async_copy(rl1, g_later.at[pl.ds(b, CH_F)], sems).wait()

    def f_pair(k):
        r0 = 2 * k
        b0 = bf + r0 * CH_F
        b1 = b0 + CH_F

        @pl.when(k > 0)
        def _():
            f_drain(b0)

        d1 = pltpu.async_copy(te.at[idxe_v.at[r0]], re0, semg0)
        d2 = pltpu.async_copy(tl.at[idxl_v.at[r0]], rl0, semg0)
        d3 = pltpu.async_copy(te.at[idxe_v.at[r0 + 1]], re1, semg1)
        d4 = pltpu.async_copy(tl.at[idxl_v.at[r0 + 1]], rl1, semg1)
        d1.wait()
        d2.wait()
        pltpu.async_copy(re0, g_early.at[pl.ds(b0, CH_F)], sems)
        pltpu.async_copy(rl0, g_later.at[pl.ds(b0, CH_F)], sems)
        d3.wait()
        d4.wait()
        pltpu.async_copy(re1, g_early.at[pl.ds(b1, CH_F)], sems)
        pltpu.async_copy(rl1, g_later.at[pl.ds(b1, CH_F)], sems)

    def f_single(i):
        b = bf + i * CH_F
        pltpu.async_copy(te.at[idxe_v.at[i]], re0, semg0).wait()
        pltpu.async_copy(re0, g_early.at[pl.ds(b, CH_F)], sems).wait()
        pltpu.async_copy(tl.at[idxl_v.at[i]], rl0, semg0).wait()
        pltpu.async_copy(rl0, g_later.at[pl.ds(b, CH_F)], sems).wait()

    _pair_loop(NF, f_pair, f_single, lambda: f_drain(bf), lambda: None)


@functools.cache
def _sc_gather():
    return pl.kernel(
        _sc_gather_body,
        out_type=(
            jax.ShapeDtypeStruct((E, 64), jnp.float32),
            jax.ShapeDtypeStruct((E, 64), jnp.float32),
            jax.ShapeDtypeStruct((ESF, 32), jnp.float32),
            jax.ShapeDtypeStruct((ESF, 32), jnp.float32),
        ),
        mesh=plsc.VectorSubcoreMesh(core_axis_name="c", subcore_axis_name="s",
                                    num_cores=NC, num_subcores=NS),
        scratch_types=[
            pltpu.VMEM((NT, CH_T), jnp.int32),
            pltpu.VMEM((NT, CH_T), jnp.int32),
            pltpu.VMEM((NF, CH_F), jnp.int32),
            pltpu.VMEM((NF, CH_F), jnp.int32),
            pltpu.VMEM((CH_T, 64), jnp.float32),
            pltpu.VMEM((CH_T, 64), jnp.float32),
            pltpu.VMEM((CH_T, 64), jnp.float32),
            pltpu.VMEM((CH_T, 64), jnp.float32),
            pltpu.VMEM((CH_F, 32), jnp.float32),
            pltpu.VMEM((CH_F, 32), jnp.float32),
            pltpu.VMEM((CH_F, 32), jnp.float32),
            pltpu.VMEM((CH_F, 32), jnp.float32),
            pltpu.SemaphoreType.DMA,
            pltpu.SemaphoreType.DMA,
            pltpu.SemaphoreType.DMA,
        ],
        compiler_params=pltpu.CompilerParams(use_tc_tiling_on_sc=False),
    )


def _mlp_t_body(gf_ref, gp_ref, a8_ref, w8_ref, b1_ref, w2d_ref, b2_ref, out_ref):
    q8 = jnp.dot(a8_ref[...], w8_ref[...], preferred_element_type=jnp.float32)
    q2 = q8.reshape(BE_T // 2, 128)
    h = jnp.maximum(gf_ref[...] + gp_ref[...] + q2 + b1_ref[...], 0.0)
    m = BE_T // 4
    lo = jnp.dot(h[0:m], w2d_ref[...], preferred_element_type=jnp.float32)
    hi = jnp.dot(h[m:2 * m], w2d_ref[...], preferred_element_type=jnp.float32)
    out_ref[...] = jnp.maximum(jnp.concatenate([lo, hi], axis=1) + b2_ref[...], 0.0)


def _mlp_t(g2f, g2p, attr8, w8, b1t, w2d, b2t):
    be = BE_T
    wspec = lambda shape: pl.BlockSpec(shape, lambda i: (0, 0))
    return pl.pallas_call(
        _mlp_t_body,
        grid=(E // be,),
        in_specs=[
            pl.BlockSpec((be // 2, 128), lambda i: (i, 0)),
            pl.BlockSpec((be // 2, 128), lambda i: (i, 0)),
            pl.BlockSpec((be // 8, 128), lambda i: (i, 0)),
            wspec((128, 512)), wspec((1, 128)), wspec((128, 64)), wspec((1, 128)),
        ],
        out_specs=pl.BlockSpec((be // 4, 128), lambda i: (i, 0)),
        out_shape=jax.ShapeDtypeStruct((E // 4, 128), jnp.float32),
    )(g2f, g2p, attr8, w8, b1t, w2d, b2t)


def _mlp_f_body(ge_ref, gl_ref, a8_ref, w8_ref, b1_ref, w2d_ref, b2_ref, out_ref):
    q8 = jnp.dot(a8_ref[...], w8_ref[...], preferred_element_type=jnp.float32)
    q4 = q8.reshape(BE_F // 4, 128)
    h = jnp.maximum(ge_ref[...] + gl_ref[...] + q4 + b1_ref[...], 0.0)
    m = BE_F // 8
    lo = jnp.dot(h[0:m], w2d_ref[...], preferred_element_type=jnp.float32)
    hi = jnp.dot(h[m:2 * m], w2d_ref[...], preferred_element_type=jnp.float32)
    out_ref[...] = jnp.maximum(jnp.concatenate([lo, hi], axis=1) + b2_ref[...], 0.0)


def _mlp_f(g4e, g4l, attr8, w8, b1t, w2d, b2t):
    be = BE_F
    wspec = lambda shape: pl.BlockSpec(shape, lambda i: (0, 0))
    return pl.pallas_call(
        _mlp_f_body,
        grid=(ESF // be,),
        in_specs=[
            pl.BlockSpec((be // 4, 128), lambda i: (i, 0)),
            pl.BlockSpec((be // 4, 128), lambda i: (i, 0)),
            pl.BlockSpec((be // 8, 128), lambda i: (i, 0)),
            wspec((128, 256)), wspec((1, 128)), wspec((128, 64)), wspec((1, 128)),
        ],
        out_specs=pl.BlockSpec((be // 8, 128), lambda i: (i, 0)),
        out_shape=jax.ShapeDtypeStruct((ESF // 8, 128), jnp.float32),
    )(g4e, g4l, attr8, w8, b1t, w2d, b2t)


def _sc_scatter_body(fl_t, fl_f, idxc2, idxe2, idxl2, zrows, out,
                     acc, idxc_v, idxe_v, idxl_v, fb0, fb1, ff0, ff1,
                     sem0, sem1, sem2, sem3):
    c = lax.axis_index("c")
    s = lax.axis_index("s")
    r0 = s * ZPW

    pltpu.sync_copy(zrows.at[pl.ds(r0, ZPW)], acc.at[pl.ds(r0, ZPW)])

    w = c * NS + s
    bt = w * SPW
    bf = w * FPW

    pltpu.sync_copy(idxc2.at[pl.ds(w * NSC, NSC)], idxc_v)
    pltpu.sync_copy(idxe2.at[pl.ds(w * NF, NF)], idxe_v)
    pltpu.sync_copy(idxl2.at[pl.ds(w * NF, NF)], idxl_v)
    plsc.subcore_barrier()

    pltpu.async_copy(fl_t.at[pl.ds(bt, CH_T)], fb0, sem0)
    pltpu.async_copy(fl_t.at[pl.ds(bt + CH_T, CH_T)], fb1, sem1)

    def t_pair(k, _):
        i0 = 2 * k
        b0 = bt + i0 * CH_T
        pltpu.make_async_copy(fl_t.at[pl.ds(b0, CH_T)], fb0, sem0).wait()
        pltpu.sync_copy(fb0, acc.at[idxc_v.at[i0]], add=True)

        @pl.when(k + 1 < NSC // 2)
        def _():
            pltpu.async_copy(fl_t.at[pl.ds(b0 + 2 * CH_T, CH_T)], fb0, sem0)

        pltpu.make_async_copy(fl_t.at[pl.ds(b0, CH_T)], fb1, sem1).wait()
        pltpu.sync_copy(fb1, acc.at[idxc_v.at[i0 + 1]], add=True)

        @pl.when(k + 1 < NSC // 2)
        def _():
            pltpu.async_copy(fl_t.at[pl.ds(b0 + 3 * CH_T, CH_T)], fb1, sem1)

        return 0

    lax.fori_loop(0, NSC // 2, t_pair, 0)

    pltpu.async_copy(fl_f.at[pl.ds(bf, CH_F)], ff0, sem2)
    pltpu.async_copy(fl_f.at[pl.ds(bf + CH_F, CH_F)], ff1, sem3)

    def f_pair(k, _):
        i0 = 2 * k
        b0 = bf + i0 * CH_F
        pltpu.make_async_copy(fl_f.at[pl.ds(b0, CH_F)], ff0, sem2).wait()
        pltpu.sync_copy(ff0, acc.at[idxe_v.at[i0]], add=True)
        pltpu.sync_copy(ff0, acc.at[idxl_v.at[i0]], add=True)

        @pl.when(k + 1 < NF // 2)
        def _():
            pltpu.async_copy(fl_f.at[pl.ds(b0 + 2 * CH_F, CH_F)], ff0, sem2)

        pltpu.make_async_copy(fl_f.at[pl.ds(b0, CH_F)], ff1, sem3).wait()
        pltpu.sync_copy(ff1, acc.at[idxe_v.at[i0 + 1]], add=True)
        pltpu.sync_copy(ff1, acc.at[idxl_v.at[i0 + 1]], add=True)

        @pl.when(k + 1 < NF // 2)
        def _():
            pltpu.async_copy(fl_f.at[pl.ds(b0 + 3 * CH_F, CH_F)], ff1, sem3)

        return 0

    lax.fori_loop(0, NF // 2, f_pair, 0)

    bl = bf + (NF - 1) * CH_F
    pltpu.async_copy(fl_f.at[pl.ds(bl, CH_F)], ff0, sem2).wait()
    pltpu.sync_copy(ff0, acc.at[idxe_v.at[NF - 1]], add=True)
    pltpu.sync_copy(ff0, acc.at[idxl_v.at[NF - 1]], add=True)

    plsc.subcore_barrier()
    pltpu.sync_copy(acc.at[pl.ds(r0, ZPW)], out.at[c, pl.ds(r0, ZPW)])


@functools.cache
def _sc_scatter():
    return pl.kernel(
        _sc_scatter_body,
        out_type=jax.ShapeDtypeStruct((NC, ACC_ROWS, 16), jnp.float32),
        mesh=plsc.VectorSubcoreMesh(core_axis_name="c", subcore_axis_name="s",
                                    num_cores=NC, num_subcores=NS),
        scratch_types=[
            pltpu.VMEM_SHARED((ACC_ROWS, 16), jnp.float32),
            pltpu.VMEM((NSC, CH_T), jnp.int32),
            pltpu.VMEM((NF, CH_F), jnp.int32),
            pltpu.VMEM((NF, CH_F), jnp.int32),
            pltpu.VMEM((CH_T, 16), jnp.float32),
            pltpu.VMEM((CH_T, 16), jnp.float32),
            pltpu.VMEM((CH_F, 16), jnp.float32),
            pltpu.VMEM((CH_F, 16), jnp.float32),
            pltpu.SemaphoreType.DMA,
            pltpu.SemaphoreType.DMA,
            pltpu.SemaphoreType.DMA,
            pltpu.SemaphoreType.DMA,
        ],
        compiler_params=pltpu.CompilerParams(use_tc_tiling_on_sc=False),
    )


def _final_body(p_ref, k0_ref, k1_ref, k2_ref, bt_ref, out_ref):
    r = N // 8
    tot = p_ref[0] + p_ref[1]
    acc = jnp.dot(tot[0:r], k0_ref[...], preferred_element_type=jnp.float32)
    acc += jnp.dot(tot[r:2 * r], k1_ref[...], preferred_element_type=jnp.float32)
    acc += jnp.dot(tot[2 * r:3 * r], k2_ref[...], preferred_element_type=jnp.float32)
    out_ref[...] = jnp.maximum(acc + bt_ref[...], 0.0)


def _final(partials, k0, k1, k2, btile):
    return pl.pallas_call(
        _final_body,
        out_shape=jax.ShapeDtypeStruct((N // 8, 1024), jnp.float32),
    )(partials, k0, k1, k2, btile)


def kernel(x, edge_index, edge_attr, same_frame_edge_index, same_frame_edge_attr,
           Wff1, bff1, Wff2, bff2,
           Wfb1, bfb1, Wfb2, bfb2,
           Wfr1, bfr1, Wfr2, bfr2,
           Wt1, bt1):
    f32 = jnp.float32
    past = edge_index[0]
    fut = edge_index[1]
    early = same_frame_edge_index[0]
    later = same_frame_edge_index[1]

    wcat = jnp.concatenate([
        Wff1[:D], Wfb1[D:2 * D],
        Wff1[D:2 * D], Wfb1[:D],
        Wfr1[:D], Wfr1[D:2 * D],
    ], axis=1)

    tf_, tp_, te_, tl_ = _proj(x, wcat)
    g_fut, g_past, g_early, g_later = _sc_gather()(
        tf_, tp_, te_, tl_,
        fut.reshape(E // CH_T, CH_T), past.reshape(E // CH_T, CH_T),
        early.reshape(ESF // CH_F, CH_F), later.reshape(ESF // CH_F, CH_F))

    eye8 = jnp.eye(8, dtype=f32)
    cboth = jnp.concatenate([Wff1[2 * D:], Wfb1[2 * D:]], axis=1)
    w8_t = jnp.kron(eye8, cboth)
    b1_t = jnp.tile(jnp.concatenate([bff1, bfb1]), 2)[None]
    w2d_t = jnp.kron(jnp.eye(2, dtype=f32),
                     jnp.concatenate([
                         jnp.concatenate([Wff2, jnp.zeros((H1, H2), f32)], axis=1),
                         jnp.concatenate([jnp.zeros((H1, H2), f32), Wfb2], axis=1),
                     ], axis=0))
    b2_t = jnp.tile(jnp.concatenate([bff2, bfb2]), 4)[None]

    fl_t = _mlp_t(g_fut.reshape(E // 2, 128), g_past.reshape(E // 2, 128),
                  edge_attr.reshape(E // 8, 128), w8_t, b1_t, w2d_t, b2_t)

    w8_f = jnp.kron(eye8, Wfr1[2 * D:])
    b1_f = jnp.tile(bfr1, 4)[None]
    w2d_f = jnp.kron(jnp.eye(4, dtype=f32), Wfr2)
    b2_f = jnp.tile(bfr2, 8)[None]

    fl_f = _mlp_f(g_early.reshape(ESF // 4, 128), g_later.reshape(ESF // 4, 128),
                  same_frame_edge_attr.reshape(ESF // 8, 128),
                  w8_f, b1_f, w2d_f, b2_f)

    idx_comb = jnp.take(jnp.concatenate([fut, past + N]), jnp.asarray(_T_PERM))
    idx_e2 = jnp.take(early, jnp.asarray(_F_PERM)) + 2 * N
    idx_l2 = jnp.take(later, jnp.asarray(_F_PERM)) + 2 * N
    zrows = jnp.zeros((ACC_ROWS, 16), f32)

    partials = _sc_scatter()(fl_t.reshape(2 * E, 16), fl_f.reshape(ESF, 16),
                             idx_comb.reshape(2 * E // CH_T, CH_T),
                             idx_e2.reshape(ESF // CH_F, CH_F),
                             idx_l2.reshape(ESF // CH_F, CH_F), zrows)

    k0 = jnp.kron(eye8, Wt1[0:16])
    k1 = jnp.kron(eye8, Wt1[32:48])
    k2 = jnp.kron(eye8, Wt1[16:32])
    btile = jnp.tile(bt1, 8)[None]

    out_p = _final(partials.reshape(NC, 3 * N // 8, 128), k0, k1, k2, btile)
    return out_p.reshape(N, 128)


# speedup vs baseline: 7.7426x; 1.0707x over previous
import functools

import jax
import jax.numpy as jnp
import numpy as np
from jax import lax
from jax.experimental import pallas as pl
from jax.experimental.pallas import tpu as pltpu
from jax.experimental.pallas import tpu_sc as plsc

N = 10000
E = 320000
ESF = 160000
D = 128
H1 = 32
H2 = 16

NC = 2
NS = 16
NW = NC * NS

EPW = E // NW
FPW = ESF // NW
CH_T = 80
CH_F = 40
NT = EPW // CH_T
NF = FPW // CH_F

SPW = 2 * E // NW
NSC = SPW // CH_T

BE_T = 6400
BE_F = 3200

ACC_ROWS = 3 * N
ZPW = ACC_ROWS // NS


def _temporal_perm():
    r = np.arange(2 * E, dtype=np.int64)
    R, u = r // 8, r % 8
    half, v = u // 4, u % 4
    blk, j = R // (BE_T // 4), R % (BE_T // 4)
    e = blk * BE_T + half * (BE_T // 2) + 2 * j + v // 2
    isfb = v % 2
    return (isfb * E + e).astype(np.int32)


def _frame_perm():
    r = np.arange(ESF, dtype=np.int64)
    R, u = r // 8, r % 8
    half, v = u // 4, u % 4
    blk, j = R // (BE_F // 8), R % (BE_F // 8)
    e = blk * BE_F + half * (BE_F // 2) + 4 * j + v
    return e.astype(np.int32)


_T_PERM = _temporal_perm()
_F_PERM = _frame_perm()


def _proj_body(x_ref, w_ref, tf_ref, tp_ref, te_ref, tl_ref):
    p = jnp.dot(x_ref[...], w_ref[...], preferred_element_type=jnp.float32)
    tf_ref[...] = p[:, 0:64]
    tp_ref[...] = p[:, 64:128]
    te_ref[...] = p[:, 128:160]
    tl_ref[...] = p[:, 160:192]


def _proj(x, wcat):
    return pl.pallas_call(
        _proj_body,
        out_shape=[
            jax.ShapeDtypeStruct((N, 64), jnp.float32),
            jax.ShapeDtypeStruct((N, 64), jnp.float32),
            jax.ShapeDtypeStruct((N, 32), jnp.float32),
            jax.ShapeDtypeStruct((N, 32), jnp.float32),
        ],
    )(x, wcat)


def _pair_loop(n_chunks, do_pair, do_single, drain, prime):
    prime()

    def body(k, _):
        do_pair(k)
        return 0

    lax.fori_loop(0, n_chunks // 2, body, 0)
    drain()
    if n_chunks % 2:
        do_single(n_chunks - 1)


def _sc_gather_t_body(tf, tp, fut2, past2, g_fut, g_past,
                      idxf_v, idxp_v, rf0, rf1, rp0, rp1, semg0, semg1, sems):
    wid = lax.axis_index("s") * NC + lax.axis_index("c")
    bt = wid * EPW

    pltpu.sync_copy(fut2.at[pl.ds(wid * NT, NT)], idxf_v)
    pltpu.sync_copy(past2.at[pl.ds(wid * NT, NT)], idxp_v)

    def t_drain(b):
        pltpu.make_async_copy(rf0, g_fut.at[pl.ds(b, CH_T)], sems).wait()
        pltpu.make_async_copy(rp0, g_past.at[pl.ds(b, CH_T)], sems).wait()
        pltpu.make_async_copy(rf1, g_fut.at[pl.ds(b, CH_T)], sems).wait()
        pltpu.make_async_copy(rp1, g_past.at[pl.ds(b, CH_T)], sems).wait()

    def t_pair(k):
        r0 = 2 * k
        b0 = bt + r0 * CH_T
        b1 = b0 + CH_T

        @pl.when(k > 0)
        def _():
            t_drain(b0)

        d1 = pltpu.async_copy(tf.at[idxf_v.at[r0]], rf0, semg0)
        d2 = pltpu.async_copy(tp.at[idxp_v.at[r0]], rp0, semg0)
        d3 = pltpu.async_copy(tf.at[idxf_v.at[r0 + 1]], rf1, semg1)
        d4 = pltpu.async_copy(tp.at[idxp_v.at[r0 + 1]], rp1, semg1)
        d1.wait()
        d2.wait()
        pltpu.async_copy(rf0, g_fut.at[pl.ds(b0, CH_T)], sems)
        pltpu.async_copy(rp0, g_past.at[pl.ds(b0, CH_T)], sems)
        d3.wait()
        d4.wait()
        pltpu.async_copy(rf1, g_fut.at[pl.ds(b1, CH_T)], sems)
        pltpu.async_copy(rp1, g_past.at[pl.ds(b1, CH_T)], sems)

    def t_single(i):
        b = bt + i * CH_T
        pltpu.async_copy(tf.at[idxf_v.at[i]], rf0, semg0).wait()
        pltpu.async_copy(rf0, g_fut.at[pl.ds(b, CH_T)], sems).wait()
        pltpu.async_copy(tp.at[idxp_v.at[i]], rp0, semg0).wait()
        pltpu.async_copy(rp0, g_past.at[pl.ds(b, CH_T)], sems).wait()

    _pair_loop(NT, t_pair, t_single, lambda: t_drain(bt), lambda: None)


def _sc_gather_f_body(te, tl, early2, later2, g_early, g_later,
                      idxe_v, idxl_v, re0, re1, rl0, rl1, semg0, semg1, sems):
    wid = lax.axis_index("s") * NC + lax.axis_index("c")
    bf = wid * FPW

    pltpu.sync_copy(early2.at[pl.ds(wid * NF, NF)], idxe_v)
    pltpu.sync_copy(later2.at[pl.ds(wid * NF, NF)], idxl_v)

    def f_drain(b):
        pltpu.make_async_copy(re0, g_early.at[pl.ds(b, CH_F)], sems).wait()
        pltpu.make_async_copy(rl0, g_later.at[pl.ds(b, CH_F)], sems).wait()
        pltpu.make_async_copy(re1, g_early.at[pl.ds(b, CH_F)], sems).wait()
        pltpu.make_async_copy(rl1, g_later.at[pl.ds(b, CH_F)], sems).wait()

    def f_pair(k):
        r0 = 2 * k
        b0 = bf + r0 * CH_F
        b1 = b0 + CH_F

        @pl.when(k > 0)
        def _():
            f_drain(b0)

        d1 = pltpu.async_copy(te.at[idxe_v.at[r0]], re0, semg0)
        d2 = pltpu.async_copy(tl.at[idxl_v.at[r0]], rl0, semg0)
        d3 = pltpu.async_copy(te.at[idxe_v.at[r0 + 1]], re1, semg1)
        d4 = pltpu.async_copy(tl.at[idxl_v.at[r0 + 1]], rl1, semg1)
        d1.wait()
        d2.wait()
        pltpu.async_copy(re0, g_early.at[pl.ds(b0, CH_F)], sems)
        pltpu.async_copy(rl0, g_later.at[pl.ds(b0, CH_F)], sems)
        d3.wait()
        d4.wait()
        pltpu.async_copy(re1, g_early.at[pl.ds(b1, CH_F)], sems)
        pltpu.async_copy(rl1, g_later.at[pl.ds(b1, CH_F)], sems)

    def f_single(i):
        b = bf + i * CH_F
        pltpu.async_copy(te.at[idxe_v.at[i]], re0, semg0).wait()
        pltpu.async_copy(re0, g_early.at[pl.ds(b, CH_F)], sems).wait()
        pltpu.async_copy(tl.at[idxl_v.at[i]], rl0, semg0).wait()
        pltpu.async_copy(rl0, g_later.at[pl.ds(b, CH_F)], sems).wait()

    _pair_loop(NF, f_pair, f_single, lambda: f_drain(bf), lambda: None)


@functools.cache
def _mesh():
    return plsc.VectorSubcoreMesh(core_axis_name="c", subcore_axis_name="s",
                                  num_cores=NC, num_subcores=NS)


@functools.cache
def _sc_gather_t():
    return pl.kernel(
        _sc_gather_t_body,
        out_type=(
            jax.ShapeDtypeStruct((E, 64), jnp.float32),
            jax.ShapeDtypeStruct((E, 64), jnp.float32),
        ),
        mesh=_mesh(),
        scratch_types=[
            pltpu.VMEM((NT, CH_T), jnp.int32),
            pltpu.VMEM((NT, CH_T), jnp.int32),
            pltpu.VMEM((CH_T, 64), jnp.float32),
            pltpu.VMEM((CH_T, 64), jnp.float32),
            pltpu.VMEM((CH_T, 64), jnp.float32),
            pltpu.VMEM((CH_T, 64), jnp.float32),
            pltpu.SemaphoreType.DMA,
            pltpu.SemaphoreType.DMA,
            pltpu.SemaphoreType.DMA,
        ],
        compiler_params=pltpu.CompilerParams(use_tc_tiling_on_sc=False),
    )


@functools.cache
def _sc_gather_f():
    return pl.kernel(
        _sc_gather_f_body,
        out_type=(
            jax.ShapeDtypeStruct((ESF, 32), jnp.float32),
            jax.ShapeDtypeStruct((ESF, 32), jnp.float32),
        ),
        mesh=_mesh(),
        scratch_types=[
            pltpu.VMEM((NF, CH_F), jnp.int32),
            pltpu.VMEM((NF, CH_F), jnp.int32),
            pltpu.VMEM((CH_F, 32), jnp.float32),
            pltpu.VMEM((CH_F, 32), jnp.float32),
            pltpu.VMEM((CH_F, 32), jnp.float32),
            pltpu.VMEM((CH_F, 32), jnp.float32),
            pltpu.SemaphoreType.DMA,
            pltpu.SemaphoreType.DMA,
            pltpu.SemaphoreType.DMA,
        ],
        compiler_params=pltpu.CompilerParams(use_tc_tiling_on_sc=False),
    )


def _mlp_t_body(gf_ref, gp_ref, a8_ref, w8_ref, b1_ref, w2d_ref, b2_ref, out_ref):
    q8 = jnp.dot(a8_ref[...], w8_ref[...], preferred_element_type=jnp.float32)
    q2 = q8.reshape(BE_T // 2, 128)
    h = jnp.maximum(gf_ref[...] + gp_ref[...] + q2 + b1_ref[...], 0.0)
    m = BE_T // 4
    lo = jnp.dot(h[0:m], w2d_ref[...], preferred_element_type=jnp.float32)
    hi = jnp.dot(h[m:2 * m], w2d_ref[...], preferred_element_type=jnp.float32)
    out_ref[...] = jnp.maximum(jnp.concatenate([lo, hi], axis=1) + b2_ref[...], 0.0)


def _mlp_t(g2f, g2p, attr8, w8, b1t, w2d, b2t):
    be = BE_T
    wspec = lambda shape: pl.BlockSpec(shape, lambda i: (0, 0))
    return pl.pallas_call(
        _mlp_t_body,
        grid=(E // be,),
        in_specs=[
            pl.BlockSpec((be // 2, 128), lambda i: (i, 0)),
            pl.BlockSpec((be // 2, 128), lambda i: (i, 0)),
            pl.BlockSpec((be // 8, 128), lambda i: (i, 0)),
            wspec((128, 512)), wspec((1, 128)), wspec((128, 64)), wspec((1, 128)),
        ],
        out_specs=pl.BlockSpec((be // 4, 128), lambda i: (i, 0)),
        out_shape=jax.ShapeDtypeStruct((E // 4, 128), jnp.float32),
    )(g2f, g2p, attr8, w8, b1t, w2d, b2t)


def _mlp_f_body(ge_ref, gl_ref, a8_ref, w8_ref, b1_ref, w2d_ref, b2_ref, out_ref):
    q8 = jnp.dot(a8_ref[...], w8_ref[...], preferred_element_type=jnp.float32)
    q4 = q8.reshape(BE_F // 4, 128)
    h = jnp.maximum(ge_ref[...] + gl_ref[...] + q4 + b1_ref[...], 0.0)
    m = BE_F // 8
    lo = jnp.dot(h[0:m], w2d_ref[...], preferred_element_type=jnp.float32)
    hi = jnp.dot(h[m:2 * m], w2d_ref[...], preferred_element_type=jnp.float32)
    out_ref[...] = jnp.maximum(jnp.concatenate([lo, hi], axis=1) + b2_ref[...], 0.0)


def _mlp_f(g4e, g4l, attr8, w8, b1t, w2d, b2t):
    be = BE_F
    wspec = lambda shape: pl.BlockSpec(shape, lambda i: (0, 0))
    return pl.pallas_call(
        _mlp_f_body,
        grid=(ESF // be,),
        in_specs=[
            pl.BlockSpec((be // 4, 128), lambda i: (i, 0)),
            pl.BlockSpec((be // 4, 128), lambda i: (i, 0)),
            pl.BlockSpec((be // 8, 128), lambda i: (i, 0)),
            wspec((128, 256)), wspec((1, 128)), wspec((128, 64)), wspec((1, 128)),
        ],
        out_specs=pl.BlockSpec((be // 8, 128), lambda i: (i, 0)),
        out_shape=jax.ShapeDtypeStruct((ESF // 8, 128), jnp.float32),
    )(g4e, g4l, attr8, w8, b1t, w2d, b2t)


ZPW_T = 2 * N // NS
ZPW_F = N // NS


def _sc_scatter_t_body(fl_t, idxc2, zrows, out,
                       acc, idxc_v, fb0, fb1, sem0, sem1):
    c = lax.axis_index("c")
    s = lax.axis_index("s")
    r0 = s * ZPW_T

    pltpu.sync_copy(zrows.at[pl.ds(r0, ZPW_T)], acc.at[pl.ds(r0, ZPW_T)])

    w = c * NS + s
    bt = w * SPW
    pltpu.sync_copy(idxc2.at[pl.ds(w * NSC, NSC)], idxc_v)
    plsc.subcore_barrier()

    pltpu.async_copy(fl_t.at[pl.ds(bt, CH_T)], fb0, sem0)
    pltpu.async_copy(fl_t.at[pl.ds(bt + CH_T, CH_T)], fb1, sem1)

    def t_pair(k, _):
        i0 = 2 * k
        b0 = bt + i0 * CH_T
        pltpu.make_async_copy(fl_t.at[pl.ds(b0, CH_T)], fb0, sem0).wait()
        pltpu.sync_copy(fb0, acc.at[idxc_v.at[i0]], add=True)

        @pl.when(k + 1 < NSC // 2)
        def _():
            pltpu.async_copy(fl_t.at[pl.ds(b0 + 2 * CH_T, CH_T)], fb0, sem0)

        pltpu.make_async_copy(fl_t.at[pl.ds(b0, CH_T)], fb1, sem1).wait()
        pltpu.sync_copy(fb1, acc.at[idxc_v.at[i0 + 1]], add=True)

        @pl.when(k + 1 < NSC // 2)
        def _():
            pltpu.async_copy(fl_t.at[pl.ds(b0 + 3 * CH_T, CH_T)], fb1, sem1)

        return 0

    lax.fori_loop(0, NSC // 2, t_pair, 0)
    plsc.subcore_barrier()
    pltpu.sync_copy(acc.at[pl.ds(r0, ZPW_T)], out.at[c, pl.ds(r0, ZPW_T)])


def _sc_scatter_f_body(fl_f, idxe2, idxl2, zrows, out,
                       acc, idxe_v, idxl_v, ff0, ff1, sem2, sem3):
    c = lax.axis_index("c")
    s = lax.axis_index("s")
    r0 = s * ZPW_F

    pltpu.sync_copy(zrows.at[pl.ds(r0, ZPW_F)], acc.at[pl.ds(r0, ZPW_F)])

    w = c * NS + s
    bf = w * FPW
    pltpu.sync_copy(idxe2.at[pl.ds(w * NF, NF)], idxe_v)
    pltpu.sync_copy(idxl2.at[pl.ds(w * NF, NF)], idxl_v)
    plsc.subcore_barrier()

    pltpu.async_copy(fl_f.at[pl.ds(bf, CH_F)], ff0, sem2)
    pltpu.async_copy(fl_f.at[pl.ds(bf + CH_F, CH_F)], ff1, sem3)

    def f_pair(k, _):
        i0 = 2 * k
        b0 = bf + i0 * CH_F
        pltpu.make_async_copy(fl_f.at[pl.ds(b0, CH_F)], ff0, sem2).wait()
        pltpu.sync_copy(ff0, acc.at[idxe_v.at[i0]], add=True)
        pltpu.sync_copy(ff0, acc.at[idxl_v.at[i0]], add=True)

        @pl.when(k + 1 < NF // 2)
        def _():
            pltpu.async_copy(fl_f.at[pl.ds(b0 + 2 * CH_F, CH_F)], ff0, sem2)

        pltpu.make_async_copy(fl_f.at[pl.ds(b0, CH_F)], ff1, sem3).wait()
        pltpu.sync_copy(ff1, acc.at[idxe_v.at[i0 + 1]], add=True)
        pltpu.sync_copy(ff1, acc.at[idxl_v.at[i0 + 1]], add=True)

        @pl.when(k + 1 < NF // 2)
        def _():
            pltpu.async_copy(fl_f.at[pl.ds(b0 + 3 * CH_F, CH_F)], ff1, sem3)

        return 0

    lax.fori_loop(0, NF // 2, f_pair, 0)

    bl = bf + (NF - 1) * CH_F
    pltpu.async_copy(fl_f.at[pl.ds(bl, CH_F)], ff0, sem2).wait()
    pltpu.sync_copy(ff0, acc.at[idxe_v.at[NF - 1]], add=True)
    pltpu.sync_copy(ff0, acc.at[idxl_v.at[NF - 1]], add=True)

    plsc.subcore_barrier()
    pltpu.sync_copy(acc.at[pl.ds(r0, ZPW_F)], out.at[c, pl.ds(r0, ZPW_F)])


@functools.cache
def _sc_scatter_t():
    return pl.kernel(
        _sc_scatter_t_body,
        out_type=jax.ShapeDtypeStruct((NC, 2 * N, 16), jnp.float32),
        mesh=_mesh(),
        scratch_types=[
            pltpu.VMEM_SHARED((2 * N, 16), jnp.float32),
            pltpu.VMEM((NSC, CH_T), jnp.int32),
            pltpu.VMEM((CH_T, 16), jnp.float32),
            pltpu.VMEM((CH_T, 16), jnp.float32),
            pltpu.SemaphoreType.DMA,
            pltpu.SemaphoreType.DMA,
        ],
        compiler_params=pltpu.CompilerParams(use_tc_tiling_on_sc=False),
    )


@functools.cache
def _sc_scatter_f():
    return pl.kernel(
        _sc_scatter_f_body,
        out_type=jax.ShapeDtypeStruct((NC, N, 16), jnp.float32),
        mesh=_mesh(),
        scratch_types=[
            pltpu.VMEM_SHARED((N, 16), jnp.float32),
            pltpu.VMEM((NF, CH_F), jnp.int32),
            pltpu.VMEM((NF, CH_F), jnp.int32),
            pltpu.VMEM((CH_F, 16), jnp.float32),
            pltpu.VMEM((CH_F, 16), jnp.float32),
            pltpu.SemaphoreType.DMA,
            pltpu.SemaphoreType.DMA,
        ],
        compiler_params=pltpu.CompilerParams(use_tc_tiling_on_sc=False),
    )


def _final_body(pt_ref, pf_ref, k0_ref, k1_ref, k2_ref, bt_ref, out_ref):
    r = N // 8
    tot = pt_ref[0] + pt_ref[1]
    fr = pf_ref[0] + pf_ref[1]
    acc = jnp.dot(tot[0:r], k0_ref[...], preferred_element_type=jnp.float32)
    acc += jnp.dot(tot[r:2 * r], k1_ref[...], preferred_element_type=jnp.float32)
    acc += jnp.dot(fr, k2_ref[...], preferred_element_type=jnp.float32)
    out_ref[...] = jnp.maximum(acc + bt_ref[...], 0.0)


def _final(pt, pf, k0, k1, k2, btile):
    return pl.pallas_call(
        _final_body,
        out_shape=jax.ShapeDtypeStruct((N // 8, 1024), jnp.float32),
    )(pt, pf, k0, k1, k2, btile)


def kernel(x, edge_index, edge_attr, same_frame_edge_index, same_frame_edge_attr,
           Wff1, bff1, Wff2, bff2,
           Wfb1, bfb1, Wfb2, bfb2,
           Wfr1, bfr1, Wfr2, bfr2,
           Wt1, bt1):
    f32 = jnp.float32
    past = edge_index[0]
    fut = edge_index[1]
    early = same_frame_edge_index[0]
    later = same_frame_edge_index[1]

    wcat = jnp.concatenate([
        Wff1[:D], Wfb1[D:2 * D],
        Wff1[D:2 * D], Wfb1[:D],
        Wfr1[:D], Wfr1[D:2 * D],
    ], axis=1)

    tf_, tp_, te_, tl_ = _proj(x, wcat)
    g_fut, g_past = _sc_gather_t()(
        tf_, tp_, fut.reshape(E // CH_T, CH_T), past.reshape(E // CH_T, CH_T))
    g_early, g_later = _sc_gather_f()(
        te_, tl_, early.reshape(ESF // CH_F, CH_F), later.reshape(ESF // CH_F, CH_F))

    eye8 = jnp.eye(8, dtype=f32)
    cboth = jnp.concatenate([Wff1[2 * D:], Wfb1[2 * D:]], axis=1)
    w8_t = jnp.kron(eye8, cboth)
    b1_t = jnp.tile(jnp.concatenate([bff1, bfb1]), 2)[None]
    w2d_t = jnp.kron(jnp.eye(2, dtype=f32),
                     jnp.concatenate([
                         jnp.concatenate([Wff2, jnp.zeros((H1, H2), f32)], axis=1),
                         jnp.concatenate([jnp.zeros((H1, H2), f32), Wfb2], axis=1),
                     ], axis=0))
    b2_t = jnp.tile(jnp.concatenate([bff2, bfb2]), 4)[None]

    fl_t = _mlp_t(g_fut.reshape(E // 2, 128), g_past.reshape(E // 2, 128),
                  edge_attr.reshape(E // 8, 128), w8_t, b1_t, w2d_t, b2_t)

    w8_f = jnp.kron(eye8, Wfr1[2 * D:])
    b1_f = jnp.tile(bfr1, 4)[None]
    w2d_f = jnp.kron(jnp.eye(4, dtype=f32), Wfr2)
    b2_f = jnp.tile(bfr2, 8)[None]

    fl_f = _mlp_f(g_early.reshape(ESF // 4, 128), g_later.reshape(ESF // 4, 128),
                  same_frame_edge_attr.reshape(ESF // 8, 128),
                  w8_f, b1_f, w2d_f, b2_f)

    idx_comb = jnp.take(jnp.concatenate([fut, past + N]), jnp.asarray(_T_PERM))
    idx_e2 = jnp.take(early, jnp.asarray(_F_PERM))
    idx_l2 = jnp.take(later, jnp.asarray(_F_PERM))
    zrows_t = jnp.zeros((2 * N, 16), f32)
    zrows_f = jnp.zeros((N, 16), f32)

    pt = _sc_scatter_t()(fl_t.reshape(2 * E, 16),
                         idx_comb.reshape(2 * E // CH_T, CH_T), zrows_t)
    pf = _sc_scatter_f()(fl_f.reshape(ESF, 16),
                         idx_e2.reshape(ESF // CH_F, CH_F),
                         idx_l2.reshape(ESF // CH_F, CH_F), zrows_f)

    k0 = jnp.kron(eye8, Wt1[0:16])
    k1 = jnp.kron(eye8, Wt1[32:48])
    k2 = jnp.kron(eye8, Wt1[16:32])
    btile = jnp.tile(bt1, 8)[None]

    out_p = _final(pt.reshape(NC, 2 * N // 8, 128), pf.reshape(NC, N // 8, 128),
                   k0, k1, k2, btile)
    return out_p.reshape(N, 128)


# speedup vs baseline: 7.8637x; 1.0156x over previous
import functools

import jax
import jax.numpy as jnp
import numpy as np
from jax import lax
from jax.experimental import pallas as pl
from jax.experimental.pallas import tpu as pltpu
from jax.experimental.pallas import tpu_sc as plsc

N = 10000
E = 320000
ESF = 160000
D = 128
H1 = 32
H2 = 16

NC = 2
NS = 16
NW = NC * NS

EPW = E // NW
FPW = ESF // NW
CH_T = 80
CH_F = 40
NT = EPW // CH_T
NF = FPW // CH_F

SPW = 2 * E // NW
NSC = SPW // CH_T

BE_T = 6400
BE_F = 3200

ACC_ROWS = 3 * N
ZPW = ACC_ROWS // NS


def _temporal_perm():
    r = np.arange(2 * E, dtype=np.int64)
    R, u = r // 8, r % 8
    half, v = u // 4, u % 4
    blk, j = R // (BE_T // 4), R % (BE_T // 4)
    e = blk * BE_T + half * (BE_T // 2) + 2 * j + v // 2
    isfb = v % 2
    return (isfb * E + e).astype(np.int32)


def _frame_perm():
    r = np.arange(ESF, dtype=np.int64)
    R, u = r // 8, r % 8
    half, v = u // 4, u % 4
    blk, j = R // (BE_F // 8), R % (BE_F // 8)
    e = blk * BE_F + half * (BE_F // 2) + 4 * j + v
    return e.astype(np.int32)


_T_PERM = _temporal_perm()
_F_PERM = _frame_perm()


def _proj_body(x_ref, w_ref, tf_ref, tp_ref, te_ref, tl_ref):
    p = jnp.dot(x_ref[...], w_ref[...], preferred_element_type=jnp.float32)
    tf_ref[...] = p[:, 0:64]
    tp_ref[...] = p[:, 64:128]
    te_ref[...] = p[:, 128:160]
    tl_ref[...] = p[:, 160:192]


def _proj(x, wcat):
    return pl.pallas_call(
        _proj_body,
        out_shape=[
            jax.ShapeDtypeStruct((N, 64), jnp.float32),
            jax.ShapeDtypeStruct((N, 64), jnp.float32),
            jax.ShapeDtypeStruct((N, 32), jnp.float32),
            jax.ShapeDtypeStruct((N, 32), jnp.float32),
        ],
    )(x, wcat)


def _pair_loop(n_chunks, do_pair, do_single, drain, prime):
    prime()

    def body(k, _):
        do_pair(k)
        return 0

    lax.fori_loop(0, n_chunks // 2, body, 0)
    drain()
    if n_chunks % 2:
        do_single(n_chunks - 1)


def _sc_gather_t_body(tf, tp, fut2, past2, g_fut, g_past,
                      idxf_v, idxp_v, rf0, rf1, rf2, rf3, rp0, rp1, rp2, rp3,
                      semg0, semg1, semg2, semg3, sems):
    wid = lax.axis_index("s") * NC + lax.axis_index("c")
    bt = wid * EPW

    pltpu.sync_copy(fut2.at[pl.ds(wid * NT, NT)], idxf_v)
    pltpu.sync_copy(past2.at[pl.ds(wid * NT, NT)], idxp_v)

    rfs = (rf0, rf1, rf2, rf3)
    rps = (rp0, rp1, rp2, rp3)
    gsems = (semg0, semg1, semg2, semg3)

    def t_drain(b):
        for q in range(4):
            pltpu.make_async_copy(rfs[q], g_fut.at[pl.ds(b, CH_T)], sems).wait()
            pltpu.make_async_copy(rps[q], g_past.at[pl.ds(b, CH_T)], sems).wait()

    def t_quad(k):
        r0 = 4 * k
        b0 = bt + r0 * CH_T

        @pl.when(k > 0)
        def _():
            t_drain(b0)

        ds_ = []
        for q in range(4):
            ds_.append(pltpu.async_copy(tf.at[idxf_v.at[r0 + q]], rfs[q], gsems[q]))
            ds_.append(pltpu.async_copy(tp.at[idxp_v.at[r0 + q]], rps[q], gsems[q]))
        for q in range(4):
            ds_[2 * q].wait()
            ds_[2 * q + 1].wait()
            bq = b0 + q * CH_T
            pltpu.async_copy(rfs[q], g_fut.at[pl.ds(bq, CH_T)], sems)
            pltpu.async_copy(rps[q], g_past.at[pl.ds(bq, CH_T)], sems)

    def body(k, _):
        t_quad(k)
        return 0

    lax.fori_loop(0, NT // 4, body, 0)
    t_drain(bt)

    i = NT - 1
    b = bt + i * CH_T
    pltpu.async_copy(tf.at[idxf_v.at[i]], rf0, semg0).wait()
    pltpu.async_copy(rf0, g_fut.at[pl.ds(b, CH_T)], sems).wait()
    pltpu.async_copy(tp.at[idxp_v.at[i]], rp0, semg0).wait()
    pltpu.async_copy(rp0, g_past.at[pl.ds(b, CH_T)], sems).wait()


def _sc_gather_f_body(te, tl, early2, later2, g_early, g_later,
                      idxe_v, idxl_v, re0, re1, rl0, rl1, semg0, semg1, sems):
    wid = lax.axis_index("s") * NC + lax.axis_index("c")
    bf = wid * FPW

    pltpu.sync_copy(early2.at[pl.ds(wid * NF, NF)], idxe_v)
    pltpu.sync_copy(later2.at[pl.ds(wid * NF, NF)], idxl_v)

    def f_drain(b):
        pltpu.make_async_copy(re0, g_early.at[pl.ds(b, CH_F)], sems).wait()
        pltpu.make_async_copy(rl0, g_later.at[pl.ds(b, CH_F)], sems).wait()
        pltpu.make_async_copy(re1, g_early.at[pl.ds(b, CH_F)], sems).wait()
        pltpu.make_async_copy(rl1, g_later.at[pl.ds(b, CH_F)], sems).wait()

    def f_pair(k):
        r0 = 2 * k
        b0 = bf + r0 * CH_F
        b1 = b0 + CH_F

        @pl.when(k > 0)
        def _():
            f_drain(b0)

        d1 = pltpu.async_copy(te.at[idxe_v.at[r0]], re0, semg0)
        d2 = pltpu.async_copy(tl.at[idxl_v.at[r0]], rl0, semg0)
        d3 = pltpu.async_copy(te.at[idxe_v.at[r0 + 1]], re1, semg1)
        d4 = pltpu.async_copy(tl.at[idxl_v.at[r0 + 1]], rl1, semg1)
        d1.wait()
        d2.wait()
        pltpu.async_copy(re0, g_early.at[pl.ds(b0, CH_F)], sems)
        pltpu.async_copy(rl0, g_later.at[pl.ds(b0, CH_F)], sems)
        d3.wait()
        d4.wait()
        pltpu.async_copy(re1, g_early.at[pl.ds(b1, CH_F)], sems)
        pltpu.async_copy(rl1, g_later.at[pl.ds(b1, CH_F)], sems)

    def f_single(i):
        b = bf + i * CH_F
        pltpu.async_copy(te.at[idxe_v.at[i]], re0, semg0).wait()
        pltpu.async_copy(re0, g_early.at[pl.ds(b, CH_F)], sems).wait()
        pltpu.async_copy(tl.at[idxl_v.at[i]], rl0, semg0).wait()
        pltpu.async_copy(rl0, g_later.at[pl.ds(b, CH_F)], sems).wait()

    _pair_loop(NF, f_pair, f_single, lambda: f_drain(bf), lambda: None)


@functools.cache
def _mesh():
    return plsc.VectorSubcoreMesh(core_axis_name="c", subcore_axis_name="s",
                                  num_cores=NC, num_subcores=NS)


@functools.cache
def _sc_gather_t():
    return pl.kernel(
        _sc_gather_t_body,
        out_type=(
            jax.ShapeDtypeStruct((E, 64), jnp.float32),
            jax.ShapeDtypeStruct((E, 64), jnp.float32),
        ),
        mesh=_mesh(),
        scratch_types=[
            pltpu.VMEM((NT, CH_T), jnp.int32),
            pltpu.VMEM((NT, CH_T), jnp.int32),
            pltpu.VMEM((CH_T, 64), jnp.float32),
            pltpu.VMEM((CH_T, 64), jnp.float32),
            pltpu.VMEM((CH_T, 64), jnp.float32),
            pltpu.VMEM((CH_T, 64), jnp.float32),
            pltpu.VMEM((CH_T, 64), jnp.float32),
            pltpu.VMEM((CH_T, 64), jnp.float32),
            pltpu.VMEM((CH_T, 64), jnp.float32),
            pltpu.VMEM((CH_T, 64), jnp.float32),
            pltpu.SemaphoreType.DMA,
            pltpu.SemaphoreType.DMA,
            pltpu.SemaphoreType.DMA,
            pltpu.SemaphoreType.DMA,
            pltpu.SemaphoreType.DMA,
        ],
        compiler_params=pltpu.CompilerParams(use_tc_tiling_on_sc=False),
    )


@functools.cache
def _sc_gather_f():
    return pl.kernel(
        _sc_gather_f_body,
        out_type=(
            jax.ShapeDtypeStruct((ESF, 32), jnp.float32),
            jax.ShapeDtypeStruct((ESF, 32), jnp.float32),
        ),
        mesh=_mesh(),
        scratch_types=[
            pltpu.VMEM((NF, CH_F), jnp.int32),
            pltpu.VMEM((NF, CH_F), jnp.int32),
            pltpu.VMEM((CH_F, 32), jnp.float32),
            pltpu.VMEM((CH_F, 32), jnp.float32),
            pltpu.VMEM((CH_F, 32), jnp.float32),
            pltpu.VMEM((CH_F, 32), jnp.float32),
            pltpu.SemaphoreType.DMA,
            pltpu.SemaphoreType.DMA,
            pltpu.SemaphoreType.DMA,
        ],
        compiler_params=pltpu.CompilerParams(use_tc_tiling_on_sc=False),
    )


def _mlp_t_body(gf_ref, gp_ref, a8_ref, w8_ref, b1_ref, w2d_ref, b2_ref, out_ref):
    q8 = jnp.dot(a8_ref[...], w8_ref[...], preferred_element_type=jnp.float32)
    q2 = q8.reshape(BE_T // 2, 128)
    h = jnp.maximum(gf_ref[...] + gp_ref[...] + q2 + b1_ref[...], 0.0)
    m = BE_T // 4
    lo = jnp.dot(h[0:m], w2d_ref[...], preferred_element_type=jnp.float32)
    hi = jnp.dot(h[m:2 * m], w2d_ref[...], preferred_element_type=jnp.float32)
    out_ref[...] = jnp.maximum(jnp.concatenate([lo, hi], axis=1) + b2_ref[...], 0.0)


def _mlp_t(g2f, g2p, attr8, w8, b1t, w2d, b2t):
    be = BE_T
    wspec = lambda shape: pl.BlockSpec(shape, lambda i: (0, 0))
    return pl.pallas_call(
        _mlp_t_body,
        grid=(E // be,),
        in_specs=[
            pl.BlockSpec((be // 2, 128), lambda i: (i, 0)),
            pl.BlockSpec((be // 2, 128), lambda i: (i, 0)),
            pl.BlockSpec((be // 8, 128), lambda i: (i, 0)),
            wspec((128, 512)), wspec((1, 128)), wspec((128, 64)), wspec((1, 128)),
        ],
        out_specs=pl.BlockSpec((be // 4, 128), lambda i: (i, 0)),
        out_shape=jax.ShapeDtypeStruct((E // 4, 128), jnp.float32),
    )(g2f, g2p, attr8, w8, b1t, w2d, b2t)


def _mlp_f_body(ge_ref, gl_ref, a8_ref, w8_ref, b1_ref, w2d_ref, b2_ref, out_ref):
    q8 = jnp.dot(a8_ref[...], w8_ref[...], preferred_element_type=jnp.float32)
    q4 = q8.reshape(BE_F // 4, 128)
    h = jnp.maximum(ge_ref[...] + gl_ref[...] + q4 + b1_ref[...], 0.0)
    m = BE_F // 8
    lo = jnp.dot(h[0:m], w2d_ref[...], preferred_element_type=jnp.float32)
    hi = jnp.dot(h[m:2 * m], w2d_ref[...], preferred_element_type=jnp.float32)
    out_ref[...] = jnp.maximum(jnp.concatenate([lo, hi], axis=1) + b2_ref[...], 0.0)


def _mlp_f(g4e, g4l, attr8, w8, b1t, w2d, b2t):
    be = BE_F
    wspec = lambda shape: pl.BlockSpec(shape, lambda i: (0, 0))
    return pl.pallas_call(
        _mlp_f_body,
        grid=(ESF // be,),
        in_specs=[
            pl.BlockSpec((be // 4, 128), lambda i: (i, 0)),
            pl.BlockSpec((be // 4, 128), lambda i: (i, 0)),
            pl.BlockSpec((be // 8, 128), lambda i: (i, 0)),
            wspec((128, 256)), wspec((1, 128)), wspec((128, 64)), wspec((1, 128)),
        ],
        out_specs=pl.BlockSpec((be // 8, 128), lambda i: (i, 0)),
        out_shape=jax.ShapeDtypeStruct((ESF // 8, 128), jnp.float32),
    )(g4e, g4l, attr8, w8, b1t, w2d, b2t)


ZPW_T = 2 * N // NS
ZPW_F = N // NS


def _sc_scatter_t_body(fl_t, idxc2, zrows, out,
                       acc, idxc_v, fb0, fb1, sem0, sem1):
    c = lax.axis_index("c")
    s = lax.axis_index("s")
    r0 = s * ZPW_T

    pltpu.sync_copy(zrows.at[pl.ds(r0, ZPW_T)], acc.at[pl.ds(r0, ZPW_T)])

    w = c * NS + s
    bt = w * SPW
    pltpu.sync_copy(idxc2.at[pl.ds(w * NSC, NSC)], idxc_v)
    plsc.subcore_barrier()

    pltpu.async_copy(fl_t.at[pl.ds(bt, CH_T)], fb0, sem0)
    pltpu.async_copy(fl_t.at[pl.ds(bt + CH_T, CH_T)], fb1, sem1)

    def t_pair(k, _):
        i0 = 2 * k
        b0 = bt + i0 * CH_T
        pltpu.make_async_copy(fl_t.at[pl.ds(b0, CH_T)], fb0, sem0).wait()
        pltpu.sync_copy(fb0, acc.at[idxc_v.at[i0]], add=True)

        @pl.when(k + 1 < NSC // 2)
        def _():
            pltpu.async_copy(fl_t.at[pl.ds(b0 + 2 * CH_T, CH_T)], fb0, sem0)

        pltpu.make_async_copy(fl_t.at[pl.ds(b0, CH_T)], fb1, sem1).wait()
        pltpu.sync_copy(fb1, acc.at[idxc_v.at[i0 + 1]], add=True)

        @pl.when(k + 1 < NSC // 2)
        def _():
            pltpu.async_copy(fl_t.at[pl.ds(b0 + 3 * CH_T, CH_T)], fb1, sem1)

        return 0

    lax.fori_loop(0, NSC // 2, t_pair, 0)
    plsc.subcore_barrier()
    pltpu.sync_copy(acc.at[pl.ds(r0, ZPW_T)], out.at[c, pl.ds(r0, ZPW_T)])


def _sc_scatter_f_body(fl_f, idxe2, idxl2, zrows, out,
                       acc, idxe_v, idxl_v, ff0, ff1, sem2, sem3):
    c = lax.axis_index("c")
    s = lax.axis_index("s")
    r0 = s * ZPW_F

    pltpu.sync_copy(zrows.at[pl.ds(r0, ZPW_F)], acc.at[pl.ds(r0, ZPW_F)])

    w = c * NS + s
    bf = w * FPW
    pltpu.sync_copy(idxe2.at[pl.ds(w * NF, NF)], idxe_v)
    pltpu.sync_copy(idxl2.at[pl.ds(w * NF, NF)], idxl_v)
    plsc.subcore_barrier()

    pltpu.async_copy(fl_f.at[pl.ds(bf, CH_F)], ff0, sem2)
    pltpu.async_copy(fl_f.at[pl.ds(bf + CH_F, CH_F)], ff1, sem3)

    def f_pair(k, _):
        i0 = 2 * k
        b0 = bf + i0 * CH_F
        pltpu.make_async_copy(fl_f.at[pl.ds(b0, CH_F)], ff0, sem2).wait()
        pltpu.sync_copy(ff0, acc.at[idxe_v.at[i0]], add=True)
        pltpu.sync_copy(ff0, acc.at[idxl_v.at[i0]], add=True)

        @pl.when(k + 1 < NF // 2)
        def _():
            pltpu.async_copy(fl_f.at[pl.ds(b0 + 2 * CH_F, CH_F)], ff0, sem2)

        pltpu.make_async_copy(fl_f.at[pl.ds(b0, CH_F)], ff1, sem3).wait()
        pltpu.sync_copy(ff1, acc.at[idxe_v.at[i0 + 1]], add=True)
        pltpu.sync_copy(ff1, acc.at[idxl_v.at[i0 + 1]], add=True)

        @pl.when(k + 1 < NF // 2)
        def _():
            pltpu.async_copy(fl_f.at[pl.ds(b0 + 3 * CH_F, CH_F)], ff1, sem3)

        return 0

    lax.fori_loop(0, NF // 2, f_pair, 0)

    bl = bf + (NF - 1) * CH_F
    pltpu.async_copy(fl_f.at[pl.ds(bl, CH_F)], ff0, sem2).wait()
    pltpu.sync_copy(ff0, acc.at[idxe_v.at[NF - 1]], add=True)
    pltpu.sync_copy(ff0, acc.at[idxl_v.at[NF - 1]], add=True)

    plsc.subcore_barrier()
    pltpu.sync_copy(acc.at[pl.ds(r0, ZPW_F)], out.at[c, pl.ds(r0, ZPW_F)])


@functools.cache
def _sc_scatter_t():
    return pl.kernel(
        _sc_scatter_t_body,
        out_type=jax.ShapeDtypeStruct((NC, 2 * N, 16), jnp.float32),
        mesh=_mesh(),
        scratch_types=[
            pltpu.VMEM_SHARED((2 * N, 16), jnp.float32),
            pltpu.VMEM((NSC, CH_T), jnp.int32),
            pltpu.VMEM((CH_T, 16), jnp.float32),
            pltpu.VMEM((CH_T, 16), jnp.float32),
            pltpu.SemaphoreType.DMA,
            pltpu.SemaphoreType.DMA,
        ],
        compiler_params=pltpu.CompilerParams(use_tc_tiling_on_sc=False),
    )


@functools.cache
def _sc_scatter_f():
    return pl.kernel(
        _sc_scatter_f_body,
        out_type=jax.ShapeDtypeStruct((NC, N, 16), jnp.float32),
        mesh=_mesh(),
        scratch_types=[
            pltpu.VMEM_SHARED((N, 16), jnp.float32),
            pltpu.VMEM((NF, CH_F), jnp.int32),
            pltpu.VMEM((NF, CH_F), jnp.int32),
            pltpu.VMEM((CH_F, 16), jnp.float32),
            pltpu.VMEM((CH_F, 16), jnp.float32),
            pltpu.SemaphoreType.DMA,
            pltpu.SemaphoreType.DMA,
        ],
        compiler_params=pltpu.CompilerParams(use_tc_tiling_on_sc=False),
    )


def _final_body(pt_ref, pf_ref, k0_ref, k1_ref, k2_ref, bt_ref, out_ref):
    r = N // 8
    tot = pt_ref[0] + pt_ref[1]
    fr = pf_ref[0] + pf_ref[1]
    acc = jnp.dot(tot[0:r], k0_ref[...], preferred_element_type=jnp.float32)
    acc += jnp.dot(tot[r:2 * r], k1_ref[...], preferred_element_type=jnp.float32)
    acc += jnp.dot(fr, k2_ref[...], preferred_element_type=jnp.float32)
    out_ref[...] = jnp.maximum(acc + bt_ref[...], 0.0)


def _final(pt, pf, k0, k1, k2, btile):
    return pl.pallas_call(
        _final_body,
        out_shape=jax.ShapeDtypeStruct((N // 8, 1024), jnp.float32),
    )(pt, pf, k0, k1, k2, btile)


def kernel(x, edge_index, edge_attr, same_frame_edge_index, same_frame_edge_attr,
           Wff1, bff1, Wff2, bff2,
           Wfb1, bfb1, Wfb2, bfb2,
           Wfr1, bfr1, Wfr2, bfr2,
           Wt1, bt1):
    f32 = jnp.float32
    past = edge_index[0]
    fut = edge_index[1]
    early = same_frame_edge_index[0]
    later = same_frame_edge_index[1]

    wcat = jnp.concatenate([
        Wff1[:D], Wfb1[D:2 * D],
        Wff1[D:2 * D], Wfb1[:D],
        Wfr1[:D], Wfr1[D:2 * D],
    ], axis=1)

    tf_, tp_, te_, tl_ = _proj(x, wcat)
    g_fut, g_past = _sc_gather_t()(
        tf_, tp_, fut.reshape(E // CH_T, CH_T), past.reshape(E // CH_T, CH_T))
    g_early, g_later = _sc_gather_f()(
        te_, tl_, early.reshape(ESF // CH_F, CH_F), later.reshape(ESF // CH_F, CH_F))

    eye8 = jnp.eye(8, dtype=f32)
    cboth = jnp.concatenate([Wff1[2 * D:], Wfb1[2 * D:]], axis=1)
    w8_t = jnp.kron(eye8, cboth)
    b1_t = jnp.tile(jnp.concatenate([bff1, bfb1]), 2)[None]
    w2d_t = jnp.kron(jnp.eye(2, dtype=f32),
                     jnp.concatenate([
                         jnp.concatenate([Wff2, jnp.zeros((H1, H2), f32)], axis=1),
                         jnp.concatenate([jnp.zeros((H1, H2), f32), Wfb2], axis=1),
                     ], axis=0))
    b2_t = jnp.tile(jnp.concatenate([bff2, bfb2]), 4)[None]

    fl_t = _mlp_t(g_fut.reshape(E // 2, 128), g_past.reshape(E // 2, 128),
                  edge_attr.reshape(E // 8, 128), w8_t, b1_t, w2d_t, b2_t)

    w8_f = jnp.kron(eye8, Wfr1[2 * D:])
    b1_f = jnp.tile(bfr1, 4)[None]
    w2d_f = jnp.kron(jnp.eye(4, dtype=f32), Wfr2)
    b2_f = jnp.tile(bfr2, 8)[None]

    fl_f = _mlp_f(g_early.reshape(ESF // 4, 128), g_later.reshape(ESF // 4, 128),
                  same_frame_edge_attr.reshape(ESF // 8, 128),
                  w8_f, b1_f, w2d_f, b2_f)

    idx_comb = jnp.take(jnp.concatenate([fut, past + N]), jnp.asarray(_T_PERM))
    idx_e2 = jnp.take(early, jnp.asarray(_F_PERM))
    idx_l2 = jnp.take(later, jnp.asarray(_F_PERM))
    zrows_t = jnp.zeros((2 * N, 16), f32)
    zrows_f = jnp.zeros((N, 16), f32)

    pt = _sc_scatter_t()(fl_t.reshape(2 * E, 16),
                         idx_comb.reshape(2 * E // CH_T, CH_T), zrows_t)
    pf = _sc_scatter_f()(fl_f.reshape(ESF, 16),
                         idx_e2.reshape(ESF // CH_F, CH_F),
                         idx_l2.reshape(ESF // CH_F, CH_F), zrows_f)

    k0 = jnp.kron(eye8, Wt1[0:16])
    k1 = jnp.kron(eye8, Wt1[32:48])
    k2 = jnp.kron(eye8, Wt1[16:32])
    btile = jnp.tile(bt1, 8)[None]

    out_p = _final(pt.reshape(NC, 2 * N // 8, 128), pf.reshape(NC, N // 8, 128),
                   k0, k1, k2, btile)
    return out_p.reshape(N, 128)
